# Initial kernel scaffold; baseline (speedup 1.0000x reference)
import functools

import jax
import jax.numpy as jnp
from jax import lax
from jax.experimental import pallas as pl
from jax.experimental.pallas import tpu as pltpu
from jax.experimental.pallas import tpu_sc as plsc

N_CIR, N_DIS = 663, 100
E_CC, E_DD = 10608, 1600
NS = 16
QE = 768
NCH = QE // 128
CC_SZ = N_CIR * N_CIR
CC_Q = 27480
DD_OFF = NS * CC_Q
DD_SZ = N_DIS * N_DIS
DD_Q = 632
BUF = DD_OFF + NS * DD_Q
Z_Q = BUF // NS


@functools.partial(
    pl.kernel,
    out_type=jax.ShapeDtypeStruct((BUF,), jnp.float32),
    mesh=plsc.VectorSubcoreMesh(core_axis_name="c", subcore_axis_name="s"),
    scratch_types=[
        pltpu.VMEM((QE,), jnp.int32),
        pltpu.VMEM((QE,), jnp.int32),
        pltpu.VMEM((NCH, 128), jnp.int32),
        pltpu.VMEM((NCH, 128), jnp.int32),
        pltpu.VMEM((NCH, 128), jnp.float32),
        pltpu.VMEM_SHARED((BUF,), jnp.float32),
        pltpu.SemaphoreType.DMA,
    ],
)
def _sc_build_adj(m_all, src_all, dst_all, zeros_hbm, out_hbm,
                  src_v, dst_v, idxg_v, idxs_v, w_v, shared, sem):
    c = lax.axis_index("c")
    s = lax.axis_index("s")
    pltpu.sync_copy(zeros_hbm, shared.at[pl.ds(s * Z_Q, Z_Q)])
    base_e = c * (NS * QE) + s * QE
    pltpu.sync_copy(src_all.at[pl.ds(base_e, QE)], src_v)
    pltpu.sync_copy(dst_all.at[pl.ds(base_e, QE)], dst_v)
    n = jnp.where(c == 0, N_CIR, N_DIS)
    gbase = jnp.where(c == 0, 0, CC_SZ)
    sbase = jnp.where(c == 0, 0, DD_OFF)
    for k in range(QE // 16):
        sv = src_v[pl.ds(k * 16, 16)]
        dv = dst_v[pl.ds(k * 16, 16)]
        j, o = k // 8, (k % 8) * 16
        idxg_v[j, pl.ds(o, 16)] = gbase + sv * n + dv
        idxs_v[j, pl.ds(o, 16)] = sbase + dv * n + sv
    copies = [
        pltpu.async_copy(m_all.at[idxg_v.at[j]], w_v.at[j], sem)
        for j in range(NCH)
    ]
    for cp in copies:
        cp.wait()
    plsc.subcore_barrier()
    for j in range(NCH):
        pltpu.sync_copy(w_v.at[j], shared.at[idxs_v.at[j]], add=True)
    plsc.subcore_barrier()

    @pl.when(c == 0)
    def _():
        pltpu.sync_copy(shared.at[pl.ds(s * CC_Q, CC_Q)],
                        out_hbm.at[pl.ds(s * CC_Q, CC_Q)])

    @pl.when(c == 1)
    def _():
        off = DD_OFF + s * DD_Q
        pltpu.sync_copy(shared.at[pl.ds(off, DD_Q)],
                        out_hbm.at[pl.ds(off, DD_Q)])


def _mm(a, b):
    return lax.dot_general(a, b, (((1,), (0,)), ((), ())),
                           preferred_element_type=jnp.float32)


def _tc_body(cc_m, dd_m, adj_cc, adj_dd,
             ec_w1, ec_b1, ec_w2, ec_b2, ec_w3, ec_b3,
             dc_w1, dc_b1, dc_w2, dc_b2, dc_w3, dc_b3,
             ed_w1, ed_b1, ed_w2, ed_b2, ed_w3, ed_b3,
             sd_w1, sd_b1, sd_w2, sd_b2, sd_w3, sd_b3,
             gc1_w, gc1_b, gc2_w, gc2_b,
             gd1_w, gd1_b, gd2_w, gd2_b,
             wcT, bc, wdT, bd,
             out_ref, cir_ref, dis_ref):
    relu = lambda x: jnp.maximum(x, 0.0)
    sig = lambda x: 1.0 / (1.0 + jnp.exp(-x))

    x_cir = relu(_mm(cc_m[...], ec_w1[...]) + ec_b1[...])
    x_cir = relu(_mm(x_cir, ec_w2[...]) + ec_b2[...])
    x_cir = relu(_mm(x_cir, ec_w3[...]) + ec_b3[...])
    x_cir = relu(_mm(x_cir, dc_w1[...]) + dc_b1[...])
    x_cir = relu(_mm(x_cir, dc_w2[...]) + dc_b2[...])
    x_cir = sig(_mm(x_cir, dc_w3[...]) + dc_b3[...])

    x_dis = relu(_mm(dd_m[...], ed_w1[...]) + ed_b1[...])
    x_dis = relu(_mm(x_dis, ed_w2[...]) + ed_b2[...])
    x_dis = relu(_mm(x_dis, ed_w3[...]) + ed_b3[...])
    x_dis = relu(_mm(x_dis, sd_w1[...]) + sd_b1[...])
    x_dis = relu(_mm(x_dis, sd_w2[...]) + sd_b2[...])
    x_dis = relu(_mm(x_dis, sd_w3[...]) + sd_b3[...])

    def norm_adj(adj, nn):
        rows = lax.broadcasted_iota(jnp.int32, (nn, nn), 0)
        cols = lax.broadcasted_iota(jnp.int32, (nn, nn), 1)
        a = adj[...] + jnp.where(rows == cols, 1.0, 0.0)
        deg = jnp.sum(a, axis=1, keepdims=True)
        dinv = jnp.where(deg > 0, lax.rsqrt(jnp.where(deg > 0, deg, 1.0)), 0.0)
        return a, dinv

    a_cc, dinv_cc = norm_adj(adj_cc, N_CIR)
    a_dd, dinv_dd = norm_adj(adj_dd, N_DIS)

    def gcn(a, dinv, x, w, b):
        h = _mm(x, w[...]) * dinv
        return relu(_mm(a, h) * dinv + b[...])

    f1c = gcn(a_cc, dinv_cc, x_cir, gc1_w, gc1_b)
    f2c = gcn(a_cc, dinv_cc, f1c, gc2_w, gc2_b)
    f1d = gcn(a_dd, dinv_dd, x_dis, gd1_w, gd1_b)
    f2d = gcn(a_dd, dinv_dd, f1d, gd2_w, gd2_b)

    cir = _mm(f1c, wcT[0:128, :]) + _mm(f2c, wcT[128:256, :]) + bc[...]
    dis = _mm(f1d, wdT[0:128, :]) + _mm(f2d, wdT[128:256, :]) + bd[...]

    cir_ref[...] = cir
    dis_ref[...] = dis
    out_ref[...] = lax.dot_general(cir, dis, (((1,), (1,)), ((), ())),
                                   preferred_element_type=jnp.float32)


def kernel(cc_data_matrix, dd_data_matrix, cc_edges, dd_edges,
           ec_w1, ec_b1, ec_w2, ec_b2, ec_w3, ec_b3,
           dc_w1, dc_b1, dc_w2, dc_b2, dc_w3, dc_b3,
           ed_w1, ed_b1, ed_w2, ed_b2, ed_w3, ed_b3,
           sd_w1, sd_b1, sd_w2, sd_b2, sd_w3, sd_b3,
           gc1_w, gc1_b, gc2_w, gc2_b,
           gd1_w, gd1_b, gd2_w, gd2_b,
           cnnc_w, cnnc_b, cnnd_w, cnnd_b):
    i32 = jnp.int32

    def pad(e, n, ne):
        npad = NS * QE - ne
        s = jnp.concatenate([e[0].astype(i32), jnp.zeros((npad,), i32)])
        d = jnp.concatenate([e[1].astype(i32), jnp.full((npad,), n, i32)])
        return s, d

    scc, dcc = pad(cc_edges, N_CIR, E_CC)
    sdd, ddd = pad(dd_edges, N_DIS, E_DD)
    src_all = jnp.concatenate([scc, sdd])
    dst_all = jnp.concatenate([dcc, ddd])
    m_all = jnp.concatenate([cc_data_matrix.reshape(-1),
                             dd_data_matrix.reshape(-1)])
    zeros = jnp.zeros((Z_Q,), jnp.float32)

    adj_flat = _sc_build_adj(m_all, src_all, dst_all, zeros)
    adj_cc = adj_flat[:CC_SZ].reshape(N_CIR, N_CIR)
    adj_dd = adj_flat[DD_OFF:DD_OFF + DD_SZ].reshape(N_DIS, N_DIS)

    biases = [b.reshape(1, -1) for b in
              (ec_b1, ec_b2, ec_b3, dc_b1, dc_b2, dc_b3,
               ed_b1, ed_b2, ed_b3, sd_b1, sd_b2, sd_b3,
               gc1_b, gc2_b, gd1_b, gd2_b)]
    (ec_b1, ec_b2, ec_b3, dc_b1, dc_b2, dc_b3,
     ed_b1, ed_b2, ed_b3, sd_b1, sd_b2, sd_b3,
     gc1_b, gc2_b, gd1_b, gd2_b) = biases
    wcT = cnnc_w.reshape(256, 256).T
    wdT = cnnd_w.reshape(256, 256).T
    bc = cnnc_b.reshape(1, -1)
    bd = cnnd_b.reshape(1, -1)

    out, cir_fea, dis_fea = pl.pallas_call(
        _tc_body,
        out_shape=[
            jax.ShapeDtypeStruct((N_CIR, N_DIS), jnp.float32),
            jax.ShapeDtypeStruct((N_CIR, 256), jnp.float32),
            jax.ShapeDtypeStruct((N_DIS, 256), jnp.float32),
        ],
    )(cc_data_matrix, dd_data_matrix, adj_cc, adj_dd,
      ec_w1, ec_b1, ec_w2, ec_b2, ec_w3, ec_b3,
      dc_w1, dc_b1, dc_w2, dc_b2, dc_w3, dc_b3,
      ed_w1, ed_b1, ed_w2, ed_b2, ed_w3, ed_b3,
      sd_w1, sd_b1, sd_w2, sd_b2, sd_w3, sd_b3,
      gc1_w, gc1_b, gc2_w, gc2_b,
      gd1_w, gd1_b, gd2_w, gd2_b,
      wcT, bc, wdT, bd)
    return out, cir_fea, dis_fea


# speedup vs baseline: 4.0787x; 4.0787x over previous
import functools

import jax
import jax.numpy as jnp
from jax import lax
from jax.experimental import pallas as pl
from jax.experimental.pallas import tpu as pltpu
from jax.experimental.pallas import tpu_sc as plsc

N_CIR, N_DIS = 663, 100
E_CC, E_DD = 10608, 1600
NS = 16
QE = 768
NCH = QE // 128
CC_SZ = N_CIR * N_CIR
CC_Q = 27480
DD_OFF = NS * CC_Q
DD_SZ = N_DIS * N_DIS
DD_Q = 632
BUF = DD_OFF + NS * DD_Q
Z_Q = BUF // NS


def _sc_body(m_all, src_all, dst_all, zeros_hbm, out_hbm,
             src_v, dst_v, idxg_v, idxs_v, w_v, stage_v, shared, sem):
    c = lax.axis_index("c")
    s = lax.axis_index("s")
    pltpu.sync_copy(zeros_hbm, stage_v)
    pltpu.sync_copy(stage_v, shared.at[pl.ds(s * Z_Q, Z_Q)])
    base_e = c * (NS * QE) + s * QE
    pltpu.sync_copy(src_all.at[pl.ds(base_e, QE)], src_v)
    pltpu.sync_copy(dst_all.at[pl.ds(base_e, QE)], dst_v)
    n = jnp.where(c == 0, N_CIR, N_DIS)
    gbase = jnp.where(c == 0, 0, CC_SZ)
    sbase = jnp.where(c == 0, 0, DD_OFF)
    for k in range(QE // 16):
        sv = src_v[pl.ds(k * 16, 16)]
        dv = dst_v[pl.ds(k * 16, 16)]
        j, o = k // 8, (k % 8) * 16
        idxg_v[j, pl.ds(o, 16)] = gbase + sv * n + dv
        idxs_v[j, pl.ds(o, 16)] = sbase + dv * n + sv
    copies = [
        pltpu.async_copy(m_all.at[idxg_v.at[j]], w_v.at[j], sem)
        for j in range(NCH)
    ]
    for cp in copies:
        cp.wait()
    plsc.subcore_barrier()
    for j in range(NCH):
        pltpu.sync_copy(w_v.at[j], shared.at[idxs_v.at[j]], add=True)
    plsc.subcore_barrier()

    @pl.when(c == 0)
    def _():
        buf = stage_v.at[pl.ds(0, CC_Q)]
        pltpu.sync_copy(shared.at[pl.ds(s * CC_Q, CC_Q)], buf)
        pltpu.sync_copy(buf, out_hbm.at[pl.ds(s * CC_Q, CC_Q)])

    @pl.when(c == 1)
    def _():
        off = DD_OFF + s * DD_Q
        buf = stage_v.at[pl.ds(0, DD_Q)]
        pltpu.sync_copy(shared.at[pl.ds(off, DD_Q)], buf)
        pltpu.sync_copy(buf, out_hbm.at[pl.ds(off, DD_Q)])


@functools.cache
def _sc_build_adj():
    return pl.kernel(
        _sc_body,
        out_type=jax.ShapeDtypeStruct((BUF,), jnp.float32),
        mesh=plsc.VectorSubcoreMesh(core_axis_name="c", subcore_axis_name="s"),
        scratch_types=[
            pltpu.VMEM((QE,), jnp.int32),
            pltpu.VMEM((QE,), jnp.int32),
            pltpu.VMEM((NCH, 128), jnp.int32),
            pltpu.VMEM((NCH, 128), jnp.int32),
            pltpu.VMEM((NCH, 128), jnp.float32),
            pltpu.VMEM((Z_Q,), jnp.float32),
            pltpu.VMEM_SHARED((BUF,), jnp.float32),
            pltpu.SemaphoreType.DMA,
        ],
    )


def _mm(a, b):
    return lax.dot_general(a, b, (((1,), (0,)), ((), ())),
                           preferred_element_type=jnp.float32)


def _tc_body(cc_m, dd_m, adj_cc, adj_dd,
             ec_w1, ec_b1, ec_w2, ec_b2, ec_w3, ec_b3,
             dc_w1, dc_b1, dc_w2, dc_b2, dc_w3, dc_b3,
             ed_w1, ed_b1, ed_w2, ed_b2, ed_w3, ed_b3,
             sd_w1, sd_b1, sd_w2, sd_b2, sd_w3, sd_b3,
             gc1_w, gc1_b, gc2_w, gc2_b,
             gd1_w, gd1_b, gd2_w, gd2_b,
             wcT, bc, wdT, bd,
             out_ref, cir_ref, dis_ref):
    relu = lambda x: jnp.maximum(x, 0.0)
    sig = lambda x: 1.0 / (1.0 + jnp.exp(-x))

    x_cir = relu(_mm(cc_m[...], ec_w1[...]) + ec_b1[...])
    x_cir = relu(_mm(x_cir, ec_w2[...]) + ec_b2[...])
    x_cir = relu(_mm(x_cir, ec_w3[...]) + ec_b3[...])
    x_cir = relu(_mm(x_cir, dc_w1[...]) + dc_b1[...])
    x_cir = relu(_mm(x_cir, dc_w2[...]) + dc_b2[...])
    x_cir = sig(_mm(x_cir, dc_w3[...]) + dc_b3[...])

    x_dis = relu(_mm(dd_m[...], ed_w1[...]) + ed_b1[...])
    x_dis = relu(_mm(x_dis, ed_w2[...]) + ed_b2[...])
    x_dis = relu(_mm(x_dis, ed_w3[...]) + ed_b3[...])
    x_dis = relu(_mm(x_dis, sd_w1[...]) + sd_b1[...])
    x_dis = relu(_mm(x_dis, sd_w2[...]) + sd_b2[...])
    x_dis = relu(_mm(x_dis, sd_w3[...]) + sd_b3[...])

    def norm_adj(adj, nn):
        rows = lax.broadcasted_iota(jnp.int32, (nn, nn), 0)
        cols = lax.broadcasted_iota(jnp.int32, (nn, nn), 1)
        a = adj[...] + jnp.where(rows == cols, 1.0, 0.0)
        deg = jnp.sum(a, axis=1, keepdims=True)
        dinv = jnp.where(deg > 0, lax.rsqrt(jnp.where(deg > 0, deg, 1.0)), 0.0)
        return a, dinv

    a_cc, dinv_cc = norm_adj(adj_cc, N_CIR)
    a_dd, dinv_dd = norm_adj(adj_dd, N_DIS)

    def gcn(a, dinv, x, w, b):
        h = _mm(x, w[...]) * dinv
        return relu(_mm(a, h) * dinv + b[...])

    f1c = gcn(a_cc, dinv_cc, x_cir, gc1_w, gc1_b)
    f2c = gcn(a_cc, dinv_cc, f1c, gc2_w, gc2_b)
    f1d = gcn(a_dd, dinv_dd, x_dis, gd1_w, gd1_b)
    f2d = gcn(a_dd, dinv_dd, f1d, gd2_w, gd2_b)

    cir = _mm(f1c, wcT[0:128, :]) + _mm(f2c, wcT[128:256, :]) + bc[...]
    dis = _mm(f1d, wdT[0:128, :]) + _mm(f2d, wdT[128:256, :]) + bd[...]

    cir_ref[...] = cir
    dis_ref[...] = dis
    out_ref[...] = lax.dot_general(cir, dis, (((1,), (1,)), ((), ())),
                                   preferred_element_type=jnp.float32)


def kernel(cc_data_matrix, dd_data_matrix, cc_edges, dd_edges,
           ec_w1, ec_b1, ec_w2, ec_b2, ec_w3, ec_b3,
           dc_w1, dc_b1, dc_w2, dc_b2, dc_w3, dc_b3,
           ed_w1, ed_b1, ed_w2, ed_b2, ed_w3, ed_b3,
           sd_w1, sd_b1, sd_w2, sd_b2, sd_w3, sd_b3,
           gc1_w, gc1_b, gc2_w, gc2_b,
           gd1_w, gd1_b, gd2_w, gd2_b,
           cnnc_w, cnnc_b, cnnd_w, cnnd_b):
    i32 = jnp.int32

    def pad(e, n, ne):
        npad = NS * QE - ne
        s = jnp.concatenate([e[0].astype(i32), jnp.zeros((npad,), i32)])
        d = jnp.concatenate([e[1].astype(i32), jnp.full((npad,), n, i32)])
        return s, d

    scc, dcc = pad(cc_edges, N_CIR, E_CC)
    sdd, ddd = pad(dd_edges, N_DIS, E_DD)
    src_all = jnp.concatenate([scc, sdd])
    dst_all = jnp.concatenate([dcc, ddd])
    m_all = jnp.concatenate([cc_data_matrix.reshape(-1),
                             dd_data_matrix.reshape(-1)])
    zeros = jnp.zeros((Z_Q,), jnp.float32)

    adj_flat = _sc_build_adj()(m_all, src_all, dst_all, zeros)
    adj_cc = adj_flat[:CC_SZ].reshape(N_CIR, N_CIR)
    adj_dd = adj_flat[DD_OFF:DD_OFF + DD_SZ].reshape(N_DIS, N_DIS)

    biases = [b.reshape(1, -1) for b in
              (ec_b1, ec_b2, ec_b3, dc_b1, dc_b2, dc_b3,
               ed_b1, ed_b2, ed_b3, sd_b1, sd_b2, sd_b3,
               gc1_b, gc2_b, gd1_b, gd2_b)]
    (ec_b1, ec_b2, ec_b3, dc_b1, dc_b2, dc_b3,
     ed_b1, ed_b2, ed_b3, sd_b1, sd_b2, sd_b3,
     gc1_b, gc2_b, gd1_b, gd2_b) = biases
    wcT = cnnc_w.reshape(256, 256).T
    wdT = cnnd_w.reshape(256, 256).T
    bc = cnnc_b.reshape(1, -1)
    bd = cnnd_b.reshape(1, -1)

    out, cir_fea, dis_fea = pl.pallas_call(
        _tc_body,
        out_shape=[
            jax.ShapeDtypeStruct((N_CIR, N_DIS), jnp.float32),
            jax.ShapeDtypeStruct((N_CIR, 256), jnp.float32),
            jax.ShapeDtypeStruct((N_DIS, 256), jnp.float32),
        ],
    )(cc_data_matrix, dd_data_matrix, adj_cc, adj_dd,
      ec_w1, ec_b1, ec_w2, ec_b2, ec_w3, ec_b3,
      dc_w1, dc_b1, dc_w2, dc_b2, dc_w3, dc_b3,
      ed_w1, ed_b1, ed_w2, ed_b2, ed_w3, ed_b3,
      sd_w1, sd_b1, sd_w2, sd_b2, sd_w3, sd_b3,
      gc1_w, gc1_b, gc2_w, gc2_b,
      gd1_w, gd1_b, gd2_w, gd2_b,
      wcT, bc, wdT, bd)
    return out, cir_fea, dis_fea


# speedup vs baseline: 4.2419x; 1.0400x over previous
import functools

import jax
import jax.numpy as jnp
from jax import lax
from jax.experimental import pallas as pl
from jax.experimental.pallas import tpu as pltpu
from jax.experimental.pallas import tpu_sc as plsc

N_CIR, N_DIS = 663, 100
E_CC, E_DD = 10608, 1600
NS = 16
QE = 768
NCH = QE // 128
CC_SZ = N_CIR * N_CIR
CC_Q = 27480
DD_OFF = NS * CC_Q
DD_SZ = N_DIS * N_DIS
DD_Q = 632
BUF = DD_OFF + NS * DD_Q
Z_Q = BUF // NS


def _sc_body(m_all, src_all, dst_all, zeros_hbm, out_hbm,
             src_v, dst_v, idxg_v, idxs_v, w_v, stage_v, shared, sem):
    c = lax.axis_index("c")
    s = lax.axis_index("s")
    with jax.named_scope("sc_zero"):
        @pl.when(c == 0)
        def _():
            buf = stage_v.at[pl.ds(0, CC_Q)]
            pltpu.sync_copy(zeros_hbm.at[pl.ds(0, CC_Q)], buf)
            pltpu.sync_copy(buf, shared.at[pl.ds(s * CC_Q, CC_Q)])

        @pl.when(c == 1)
        def _():
            buf = stage_v.at[pl.ds(0, DD_Q)]
            pltpu.sync_copy(zeros_hbm.at[pl.ds(0, DD_Q)], buf)
            pltpu.sync_copy(buf, shared.at[pl.ds(DD_OFF + s * DD_Q, DD_Q)])

    with jax.named_scope("sc_edges"):
        base_e = c * (NS * QE) + s * QE
        pltpu.sync_copy(src_all.at[pl.ds(base_e, QE)], src_v)
        pltpu.sync_copy(dst_all.at[pl.ds(base_e, QE)], dst_v)
    n = jnp.where(c == 0, N_CIR, N_DIS)
    gbase = jnp.where(c == 0, 0, CC_SZ)
    sbase = jnp.where(c == 0, 0, DD_OFF)
    with jax.named_scope("sc_idx"):
        for k in range(QE // 16):
            sv = src_v[pl.ds(k * 16, 16)]
            dv = dst_v[pl.ds(k * 16, 16)]
            j, o = k // 8, (k % 8) * 16
            idxg_v[j, pl.ds(o, 16)] = gbase + sv * n + dv
            idxs_v[j, pl.ds(o, 16)] = sbase + dv * n + sv
    with jax.named_scope("sc_gather"):
        copies = [
            pltpu.async_copy(m_all.at[idxg_v.at[j]], w_v.at[j], sem)
            for j in range(NCH)
        ]
        for cp in copies:
            cp.wait()
    with jax.named_scope("sc_bar1"):
        plsc.subcore_barrier()
    with jax.named_scope("sc_scatter"):
        for j in range(NCH):
            pltpu.sync_copy(w_v.at[j], shared.at[idxs_v.at[j]], add=True)
    with jax.named_scope("sc_bar2"):
        plsc.subcore_barrier()

    with jax.named_scope("sc_out"):
        @pl.when(c == 0)
        def _():
            buf = stage_v.at[pl.ds(0, CC_Q)]
            pltpu.sync_copy(shared.at[pl.ds(s * CC_Q, CC_Q)], buf)
            pltpu.sync_copy(buf, out_hbm.at[pl.ds(s * CC_Q, CC_Q)])

        @pl.when(c == 1)
        def _():
            off = DD_OFF + s * DD_Q
            buf = stage_v.at[pl.ds(0, DD_Q)]
            pltpu.sync_copy(shared.at[pl.ds(off, DD_Q)], buf)
            pltpu.sync_copy(buf, out_hbm.at[pl.ds(off, DD_Q)])


@functools.cache
def _sc_build_adj():
    return pl.kernel(
        _sc_body,
        out_type=jax.ShapeDtypeStruct((BUF,), jnp.float32),
        mesh=plsc.VectorSubcoreMesh(core_axis_name="c", subcore_axis_name="s"),
        scratch_types=[
            pltpu.VMEM((QE,), jnp.int32),
            pltpu.VMEM((QE,), jnp.int32),
            pltpu.VMEM((NCH, 128), jnp.int32),
            pltpu.VMEM((NCH, 128), jnp.int32),
            pltpu.VMEM((NCH, 128), jnp.float32),
            pltpu.VMEM((Z_Q,), jnp.float32),
            pltpu.VMEM_SHARED((BUF,), jnp.float32),
            pltpu.SemaphoreType.DMA,
        ],
    )


def _mm(a, b):
    return lax.dot_general(a, b, (((1,), (0,)), ((), ())),
                           preferred_element_type=jnp.float32)


def _tc_body(cc_m, dd_m, adj_cc, adj_dd,
             ec_w1, ec_b1, ec_w2, ec_b2, ec_w3, ec_b3,
             dc_w1, dc_b1, dc_w2, dc_b2, dc_w3, dc_b3,
             ed_w1, ed_b1, ed_w2, ed_b2, ed_w3, ed_b3,
             sd_w1, sd_b1, sd_w2, sd_b2, sd_w3, sd_b3,
             gc1_w, gc1_b, gc2_w, gc2_b,
             gd1_w, gd1_b, gd2_w, gd2_b,
             wcT, bc, wdT, bd,
             out_ref, cir_ref, dis_ref):
    relu = lambda x: jnp.maximum(x, 0.0)
    sig = lambda x: 1.0 / (1.0 + jnp.exp(-x))

    x_cir = relu(_mm(cc_m[...], ec_w1[...]) + ec_b1[...])
    x_cir = relu(_mm(x_cir, ec_w2[...]) + ec_b2[...])
    x_cir = relu(_mm(x_cir, ec_w3[...]) + ec_b3[...])
    x_cir = relu(_mm(x_cir, dc_w1[...]) + dc_b1[...])
    x_cir = relu(_mm(x_cir, dc_w2[...]) + dc_b2[...])
    x_cir = sig(_mm(x_cir, dc_w3[...]) + dc_b3[...])

    x_dis = relu(_mm(dd_m[...], ed_w1[...]) + ed_b1[...])
    x_dis = relu(_mm(x_dis, ed_w2[...]) + ed_b2[...])
    x_dis = relu(_mm(x_dis, ed_w3[...]) + ed_b3[...])
    x_dis = relu(_mm(x_dis, sd_w1[...]) + sd_b1[...])
    x_dis = relu(_mm(x_dis, sd_w2[...]) + sd_b2[...])
    x_dis = relu(_mm(x_dis, sd_w3[...]) + sd_b3[...])

    def norm_adj(adj, nn):
        rows = lax.broadcasted_iota(jnp.int32, (nn, nn), 0)
        cols = lax.broadcasted_iota(jnp.int32, (nn, nn), 1)
        a = adj[...] + jnp.where(rows == cols, 1.0, 0.0)
        deg = jnp.sum(a, axis=1, keepdims=True)
        dinv = jnp.where(deg > 0, lax.rsqrt(jnp.where(deg > 0, deg, 1.0)), 0.0)
        return a, dinv

    a_cc, dinv_cc = norm_adj(adj_cc, N_CIR)
    a_dd, dinv_dd = norm_adj(adj_dd, N_DIS)

    def gcn(a, dinv, x, w, b):
        h = _mm(x, w[...]) * dinv
        return relu(_mm(a, h) * dinv + b[...])

    f1c = gcn(a_cc, dinv_cc, x_cir, gc1_w, gc1_b)
    f2c = gcn(a_cc, dinv_cc, f1c, gc2_w, gc2_b)
    f1d = gcn(a_dd, dinv_dd, x_dis, gd1_w, gd1_b)
    f2d = gcn(a_dd, dinv_dd, f1d, gd2_w, gd2_b)

    cir = _mm(f1c, wcT[0:128, :]) + _mm(f2c, wcT[128:256, :]) + bc[...]
    dis = _mm(f1d, wdT[0:128, :]) + _mm(f2d, wdT[128:256, :]) + bd[...]

    cir_ref[...] = cir
    dis_ref[...] = dis
    out_ref[...] = lax.dot_general(cir, dis, (((1,), (1,)), ((), ())),
                                   preferred_element_type=jnp.float32)


def kernel(cc_data_matrix, dd_data_matrix, cc_edges, dd_edges,
           ec_w1, ec_b1, ec_w2, ec_b2, ec_w3, ec_b3,
           dc_w1, dc_b1, dc_w2, dc_b2, dc_w3, dc_b3,
           ed_w1, ed_b1, ed_w2, ed_b2, ed_w3, ed_b3,
           sd_w1, sd_b1, sd_w2, sd_b2, sd_w3, sd_b3,
           gc1_w, gc1_b, gc2_w, gc2_b,
           gd1_w, gd1_b, gd2_w, gd2_b,
           cnnc_w, cnnc_b, cnnd_w, cnnd_b):
    i32 = jnp.int32

    def pad(e, n, ne):
        npad = NS * QE - ne
        s = jnp.concatenate([e[0].astype(i32), jnp.zeros((npad,), i32)])
        d = jnp.concatenate([e[1].astype(i32), jnp.full((npad,), n, i32)])
        return s, d

    scc, dcc = pad(cc_edges, N_CIR, E_CC)
    sdd, ddd = pad(dd_edges, N_DIS, E_DD)
    src_all = jnp.concatenate([scc, sdd])
    dst_all = jnp.concatenate([dcc, ddd])
    m_all = jnp.concatenate([cc_data_matrix.reshape(-1),
                             dd_data_matrix.reshape(-1)])
    zeros = jnp.zeros((CC_Q,), jnp.float32)

    adj_flat = _sc_build_adj()(m_all, src_all, dst_all, zeros)
    adj_cc = adj_flat[:CC_SZ].reshape(N_CIR, N_CIR)
    adj_dd = adj_flat[DD_OFF:DD_OFF + DD_SZ].reshape(N_DIS, N_DIS)

    biases = [b.reshape(1, -1) for b in
              (ec_b1, ec_b2, ec_b3, dc_b1, dc_b2, dc_b3,
               ed_b1, ed_b2, ed_b3, sd_b1, sd_b2, sd_b3,
               gc1_b, gc2_b, gd1_b, gd2_b)]
    (ec_b1, ec_b2, ec_b3, dc_b1, dc_b2, dc_b3,
     ed_b1, ed_b2, ed_b3, sd_b1, sd_b2, sd_b3,
     gc1_b, gc2_b, gd1_b, gd2_b) = biases
    wcT = cnnc_w.reshape(256, 256).T
    wdT = cnnd_w.reshape(256, 256).T
    bc = cnnc_b.reshape(1, -1)
    bd = cnnd_b.reshape(1, -1)

    out, cir_fea, dis_fea = pl.pallas_call(
        _tc_body,
        out_shape=[
            jax.ShapeDtypeStruct((N_CIR, N_DIS), jnp.float32),
            jax.ShapeDtypeStruct((N_CIR, 256), jnp.float32),
            jax.ShapeDtypeStruct((N_DIS, 256), jnp.float32),
        ],
    )(cc_data_matrix, dd_data_matrix, adj_cc, adj_dd,
      ec_w1, ec_b1, ec_w2, ec_b2, ec_w3, ec_b3,
      dc_w1, dc_b1, dc_w2, dc_b2, dc_w3, dc_b3,
      ed_w1, ed_b1, ed_w2, ed_b2, ed_w3, ed_b3,
      sd_w1, sd_b1, sd_w2, sd_b2, sd_w3, sd_b3,
      gc1_w, gc1_b, gc2_w, gc2_b,
      gd1_w, gd1_b, gd2_w, gd2_b,
      wcT, bc, wdT, bd)
    return out, cir_fea, dis_fea


# speedup vs baseline: 6.1653x; 1.4534x over previous
import functools

import jax
import jax.numpy as jnp
from jax import lax
from jax.experimental import pallas as pl
from jax.experimental.pallas import tpu as pltpu
from jax.experimental.pallas import tpu_sc as plsc

N_CIR, N_DIS = 663, 100
E_CC, E_DD = 10608, 1600
NS = 16
QE = 768
NCH = QE // 128
CC_SZ = N_CIR * N_CIR
CC_Q = 27480
DD_OFF = NS * CC_Q
DD_SZ = N_DIS * N_DIS
DD_Q = 632
BUF = DD_OFF + NS * DD_Q
Z_Q = BUF // NS
GB_D = CC_SZ + 7
M_LEN = GB_D + NS * DD_Q


def _sc_body(m_all, src_all, dst_all, zeros_hbm, out_hbm,
             src_v, dst_v, idxg_v, idxs_v, w_v, stage_v, shared, m_sh, sem):
    c = lax.axis_index("c")
    s = lax.axis_index("s")
    with jax.named_scope("sc_zero"):
        @pl.when(c == 0)
        def _():
            buf = stage_v.at[pl.ds(0, CC_Q)]
            pltpu.sync_copy(zeros_hbm.at[pl.ds(0, CC_Q)], buf)
            pltpu.sync_copy(buf, shared.at[pl.ds(s * CC_Q, CC_Q)])

        @pl.when(c == 1)
        def _():
            buf = stage_v.at[pl.ds(0, DD_Q)]
            pltpu.sync_copy(zeros_hbm.at[pl.ds(0, DD_Q)], buf)
            pltpu.sync_copy(buf, shared.at[pl.ds(DD_OFF + s * DD_Q, DD_Q)])

    with jax.named_scope("sc_mstage"):
        @pl.when(c == 0)
        def _():
            buf = stage_v.at[pl.ds(0, CC_Q)]
            pltpu.sync_copy(m_all.at[pl.ds(s * CC_Q, CC_Q)], buf)
            pltpu.sync_copy(buf, m_sh.at[pl.ds(s * CC_Q, CC_Q)])

        @pl.when(c == 1)
        def _():
            off = GB_D + s * DD_Q
            buf = stage_v.at[pl.ds(0, DD_Q)]
            pltpu.sync_copy(m_all.at[pl.ds(off, DD_Q)], buf)
            pltpu.sync_copy(buf, m_sh.at[pl.ds(off, DD_Q)])

    with jax.named_scope("sc_edges"):
        base_e = c * (NS * QE) + s * QE
        pltpu.sync_copy(src_all.at[pl.ds(base_e, QE)], src_v)
        pltpu.sync_copy(dst_all.at[pl.ds(base_e, QE)], dst_v)
    n = jnp.where(c == 0, N_CIR, N_DIS)
    gbase = jnp.where(c == 0, 0, GB_D)
    sbase = jnp.where(c == 0, 0, DD_OFF)
    with jax.named_scope("sc_idx"):
        for k in range(QE // 16):
            sv = src_v[pl.ds(k * 16, 16)]
            dv = dst_v[pl.ds(k * 16, 16)]
            j, o = k // 8, (k % 8) * 16
            idxg_v[j, pl.ds(o, 16)] = gbase + sv * n + dv
            idxs_v[j, pl.ds(o, 16)] = sbase + dv * n + sv
    with jax.named_scope("sc_bar1"):
        plsc.subcore_barrier()
    with jax.named_scope("sc_gather"):
        copies = [
            pltpu.async_copy(m_sh.at[idxg_v.at[j]], w_v.at[j], sem)
            for j in range(NCH)
        ]
        for cp in copies:
            cp.wait()
    with jax.named_scope("sc_scatter"):
        for j in range(NCH):
            pltpu.sync_copy(w_v.at[j], shared.at[idxs_v.at[j]], add=True)
    with jax.named_scope("sc_bar2"):
        plsc.subcore_barrier()

    with jax.named_scope("sc_out"):
        @pl.when(c == 0)
        def _():
            buf = stage_v.at[pl.ds(0, CC_Q)]
            pltpu.sync_copy(shared.at[pl.ds(s * CC_Q, CC_Q)], buf)
            pltpu.sync_copy(buf, out_hbm.at[pl.ds(s * CC_Q, CC_Q)])

        @pl.when(c == 1)
        def _():
            off = DD_OFF + s * DD_Q
            buf = stage_v.at[pl.ds(0, DD_Q)]
            pltpu.sync_copy(shared.at[pl.ds(off, DD_Q)], buf)
            pltpu.sync_copy(buf, out_hbm.at[pl.ds(off, DD_Q)])


@functools.cache
def _sc_build_adj():
    return pl.kernel(
        _sc_body,
        out_type=jax.ShapeDtypeStruct((BUF,), jnp.float32),
        mesh=plsc.VectorSubcoreMesh(core_axis_name="c", subcore_axis_name="s"),
        scratch_types=[
            pltpu.VMEM((QE,), jnp.int32),
            pltpu.VMEM((QE,), jnp.int32),
            pltpu.VMEM((NCH, 128), jnp.int32),
            pltpu.VMEM((NCH, 128), jnp.int32),
            pltpu.VMEM((NCH, 128), jnp.float32),
            pltpu.VMEM((Z_Q,), jnp.float32),
            pltpu.VMEM_SHARED((BUF,), jnp.float32),
            pltpu.VMEM_SHARED((M_LEN,), jnp.float32),
            pltpu.SemaphoreType.DMA,
        ],
    )


def _mm(a, b):
    return lax.dot_general(a, b, (((1,), (0,)), ((), ())),
                           preferred_element_type=jnp.float32)


def _tc_body(cc_m, dd_m, adj_cc, adj_dd,
             ec_w1, ec_b1, ec_w2, ec_b2, ec_w3, ec_b3,
             dc_w1, dc_b1, dc_w2, dc_b2, dc_w3, dc_b3,
             ed_w1, ed_b1, ed_w2, ed_b2, ed_w3, ed_b3,
             sd_w1, sd_b1, sd_w2, sd_b2, sd_w3, sd_b3,
             gc1_w, gc1_b, gc2_w, gc2_b,
             gd1_w, gd1_b, gd2_w, gd2_b,
             wcT, bc, wdT, bd,
             out_ref, cir_ref, dis_ref):
    relu = lambda x: jnp.maximum(x, 0.0)
    sig = lambda x: 1.0 / (1.0 + jnp.exp(-x))

    x_cir = relu(_mm(cc_m[...], ec_w1[...]) + ec_b1[...])
    x_cir = relu(_mm(x_cir, ec_w2[...]) + ec_b2[...])
    x_cir = relu(_mm(x_cir, ec_w3[...]) + ec_b3[...])
    x_cir = relu(_mm(x_cir, dc_w1[...]) + dc_b1[...])
    x_cir = relu(_mm(x_cir, dc_w2[...]) + dc_b2[...])
    x_cir = sig(_mm(x_cir, dc_w3[...]) + dc_b3[...])

    x_dis = relu(_mm(dd_m[...], ed_w1[...]) + ed_b1[...])
    x_dis = relu(_mm(x_dis, ed_w2[...]) + ed_b2[...])
    x_dis = relu(_mm(x_dis, ed_w3[...]) + ed_b3[...])
    x_dis = relu(_mm(x_dis, sd_w1[...]) + sd_b1[...])
    x_dis = relu(_mm(x_dis, sd_w2[...]) + sd_b2[...])
    x_dis = relu(_mm(x_dis, sd_w3[...]) + sd_b3[...])

    def norm_adj(adj, nn):
        rows = lax.broadcasted_iota(jnp.int32, (nn, nn), 0)
        cols = lax.broadcasted_iota(jnp.int32, (nn, nn), 1)
        a = adj[...] + jnp.where(rows == cols, 1.0, 0.0)
        deg = jnp.sum(a, axis=1, keepdims=True)
        dinv = jnp.where(deg > 0, lax.rsqrt(jnp.where(deg > 0, deg, 1.0)), 0.0)
        return a, dinv

    a_cc, dinv_cc = norm_adj(adj_cc, N_CIR)
    a_dd, dinv_dd = norm_adj(adj_dd, N_DIS)

    def gcn(a, dinv, x, w, b):
        h = _mm(x, w[...]) * dinv
        return relu(_mm(a, h) * dinv + b[...])

    f1c = gcn(a_cc, dinv_cc, x_cir, gc1_w, gc1_b)
    f2c = gcn(a_cc, dinv_cc, f1c, gc2_w, gc2_b)
    f1d = gcn(a_dd, dinv_dd, x_dis, gd1_w, gd1_b)
    f2d = gcn(a_dd, dinv_dd, f1d, gd2_w, gd2_b)

    cir = _mm(f1c, wcT[0:128, :]) + _mm(f2c, wcT[128:256, :]) + bc[...]
    dis = _mm(f1d, wdT[0:128, :]) + _mm(f2d, wdT[128:256, :]) + bd[...]

    cir_ref[...] = cir
    dis_ref[...] = dis
    out_ref[...] = lax.dot_general(cir, dis, (((1,), (1,)), ((), ())),
                                   preferred_element_type=jnp.float32)


def kernel(cc_data_matrix, dd_data_matrix, cc_edges, dd_edges,
           ec_w1, ec_b1, ec_w2, ec_b2, ec_w3, ec_b3,
           dc_w1, dc_b1, dc_w2, dc_b2, dc_w3, dc_b3,
           ed_w1, ed_b1, ed_w2, ed_b2, ed_w3, ed_b3,
           sd_w1, sd_b1, sd_w2, sd_b2, sd_w3, sd_b3,
           gc1_w, gc1_b, gc2_w, gc2_b,
           gd1_w, gd1_b, gd2_w, gd2_b,
           cnnc_w, cnnc_b, cnnd_w, cnnd_b):
    i32 = jnp.int32

    def pad(e, n, ne):
        npad = NS * QE - ne
        s = jnp.concatenate([e[0].astype(i32), jnp.zeros((npad,), i32)])
        d = jnp.concatenate([e[1].astype(i32), jnp.full((npad,), n, i32)])
        return s, d

    scc, dcc = pad(cc_edges, N_CIR, E_CC)
    sdd, ddd = pad(dd_edges, N_DIS, E_DD)
    src_all = jnp.concatenate([scc, sdd])
    dst_all = jnp.concatenate([dcc, ddd])
    m_all = jnp.concatenate([cc_data_matrix.reshape(-1),
                             jnp.zeros((GB_D - CC_SZ,), jnp.float32),
                             dd_data_matrix.reshape(-1),
                             jnp.zeros((M_LEN - GB_D - DD_SZ,), jnp.float32)])
    zeros = jnp.zeros((CC_Q,), jnp.float32)

    adj_flat = _sc_build_adj()(m_all, src_all, dst_all, zeros)
    adj_cc = adj_flat[:CC_SZ].reshape(N_CIR, N_CIR)
    adj_dd = adj_flat[DD_OFF:DD_OFF + DD_SZ].reshape(N_DIS, N_DIS)

    biases = [b.reshape(1, -1) for b in
              (ec_b1, ec_b2, ec_b3, dc_b1, dc_b2, dc_b3,
               ed_b1, ed_b2, ed_b3, sd_b1, sd_b2, sd_b3,
               gc1_b, gc2_b, gd1_b, gd2_b)]
    (ec_b1, ec_b2, ec_b3, dc_b1, dc_b2, dc_b3,
     ed_b1, ed_b2, ed_b3, sd_b1, sd_b2, sd_b3,
     gc1_b, gc2_b, gd1_b, gd2_b) = biases
    wcT = cnnc_w.reshape(256, 256).T
    wdT = cnnd_w.reshape(256, 256).T
    bc = cnnc_b.reshape(1, -1)
    bd = cnnd_b.reshape(1, -1)

    out, cir_fea, dis_fea = pl.pallas_call(
        _tc_body,
        out_shape=[
            jax.ShapeDtypeStruct((N_CIR, N_DIS), jnp.float32),
            jax.ShapeDtypeStruct((N_CIR, 256), jnp.float32),
            jax.ShapeDtypeStruct((N_DIS, 256), jnp.float32),
        ],
    )(cc_data_matrix, dd_data_matrix, adj_cc, adj_dd,
      ec_w1, ec_b1, ec_w2, ec_b2, ec_w3, ec_b3,
      dc_w1, dc_b1, dc_w2, dc_b2, dc_w3, dc_b3,
      ed_w1, ed_b1, ed_w2, ed_b2, ed_w3, ed_b3,
      sd_w1, sd_b1, sd_w2, sd_b2, sd_w3, sd_b3,
      gc1_w, gc1_b, gc2_w, gc2_b,
      gd1_w, gd1_b, gd2_w, gd2_b,
      wcT, bc, wdT, bd)
    return out, cir_fea, dis_fea


# speedup vs baseline: 8.3791x; 1.3591x over previous
import functools

import jax
import jax.numpy as jnp
from jax import lax
from jax.experimental import pallas as pl
from jax.experimental.pallas import tpu as pltpu
from jax.experimental.pallas import tpu_sc as plsc

N_CIR, N_DIS = 663, 100
E_CC, E_DD = 10608, 1600
NS = 16
QE = 768
NCH = QE // 128
QE_D = 128
CC_SZ = N_CIR * N_CIR
CC_Q = 27480
CC_QL = 27368
DD_OFF = NS * CC_Q
DD_SZ = N_DIS * N_DIS
DD_Q = 632
BUF = DD_OFF + NS * DD_Q
Z_Q = BUF // NS
GB_D = CC_SZ + 7
M_LEN = GB_D + NS * DD_Q


def _sc_body(m_cc, m_dd, m_tail, src_all, dst_all, zeros_hbm, out_hbm,
             src_v, dst_v, idxg_v, idxs_v, w_v, stage_v, shared, m_sh, sem):
    c = lax.axis_index("c")
    s = lax.axis_index("s")
    with jax.named_scope("sc_zero"):
        @pl.when(c == 0)
        def _():
            buf = stage_v.at[pl.ds(0, CC_Q)]
            pltpu.sync_copy(zeros_hbm.at[pl.ds(0, CC_Q)], buf)
            pltpu.sync_copy(buf, shared.at[pl.ds(s * CC_Q, CC_Q)])

        @pl.when(c == 1)
        def _():
            buf = stage_v.at[pl.ds(0, DD_Q)]
            pltpu.sync_copy(zeros_hbm.at[pl.ds(0, DD_Q)], buf)
            pltpu.sync_copy(buf, shared.at[pl.ds(DD_OFF + s * DD_Q, DD_Q)])

    with jax.named_scope("sc_mstage"):
        @pl.when((c == 0) & (s < NS - 1))
        def _():
            buf = stage_v.at[pl.ds(0, CC_Q)]
            pltpu.sync_copy(m_cc.at[pl.ds(s * CC_Q, CC_Q)], buf)
            pltpu.sync_copy(buf, m_sh.at[pl.ds(s * CC_Q, CC_Q)])

        @pl.when((c == 0) & (s == NS - 1))
        def _():
            buf = stage_v.at[pl.ds(0, CC_QL)]
            pltpu.sync_copy(m_cc.at[pl.ds((NS - 1) * CC_Q, CC_QL)], buf)
            pltpu.sync_copy(buf, m_sh.at[pl.ds((NS - 1) * CC_Q, CC_QL)])
            tl = stage_v.at[pl.ds(0, 8)]
            pltpu.sync_copy(m_tail, tl)
            pltpu.sync_copy(tl, m_sh.at[pl.ds(CC_SZ - 1, 8)])

        @pl.when(c == 1)
        def _():
            buf = stage_v.at[pl.ds(0, DD_Q)]
            pltpu.sync_copy(m_dd.at[pl.ds(s * DD_Q, DD_Q)], buf)
            pltpu.sync_copy(buf, m_sh.at[pl.ds(GB_D + s * DD_Q, DD_Q)])

    def edge_prep(ebase, nch, n, gbase, sbase):
        qe = nch * 128
        pltpu.sync_copy(src_all.at[pl.ds(ebase, qe)], src_v.at[pl.ds(0, qe)])
        pltpu.sync_copy(dst_all.at[pl.ds(ebase, qe)], dst_v.at[pl.ds(0, qe)])
        for k in range(qe // 16):
            sv = src_v[pl.ds(k * 16, 16)]
            dv = dst_v[pl.ds(k * 16, 16)]
            j, o = k // 8, (k % 8) * 16
            idxg_v[j, pl.ds(o, 16)] = gbase + sv * n + dv
            idxs_v[j, pl.ds(o, 16)] = sbase + dv * n + sv

    with jax.named_scope("sc_idx"):
        @pl.when(c == 0)
        def _():
            edge_prep(s * QE, NCH, N_CIR, 0, 0)

        @pl.when(c == 1)
        def _():
            edge_prep(NS * QE + s * QE_D, 1, N_DIS, GB_D, DD_OFF)

    with jax.named_scope("sc_bar1"):
        plsc.subcore_barrier()

    def gather_scatter(nch):
        copies = [
            pltpu.async_copy(m_sh.at[idxg_v.at[j]], w_v.at[j], sem)
            for j in range(nch)
        ]
        for cp in copies:
            cp.wait()
        for j in range(nch):
            pltpu.sync_copy(w_v.at[j], shared.at[idxs_v.at[j]], add=True)

    with jax.named_scope("sc_gsc"):
        @pl.when(c == 0)
        def _():
            gather_scatter(NCH)

        @pl.when(c == 1)
        def _():
            gather_scatter(1)

    with jax.named_scope("sc_bar2"):
        plsc.subcore_barrier()

    with jax.named_scope("sc_out"):
        @pl.when(c == 0)
        def _():
            buf = stage_v.at[pl.ds(0, CC_Q)]
            pltpu.sync_copy(shared.at[pl.ds(s * CC_Q, CC_Q)], buf)
            pltpu.sync_copy(buf, out_hbm.at[pl.ds(s * CC_Q, CC_Q)])

        @pl.when(c == 1)
        def _():
            off = DD_OFF + s * DD_Q
            buf = stage_v.at[pl.ds(0, DD_Q)]
            pltpu.sync_copy(shared.at[pl.ds(off, DD_Q)], buf)
            pltpu.sync_copy(buf, out_hbm.at[pl.ds(off, DD_Q)])


@functools.cache
def _sc_build_adj():
    return pl.kernel(
        _sc_body,
        out_type=jax.ShapeDtypeStruct((BUF,), jnp.float32),
        mesh=plsc.VectorSubcoreMesh(core_axis_name="c", subcore_axis_name="s"),
        scratch_types=[
            pltpu.VMEM((QE,), jnp.int32),
            pltpu.VMEM((QE,), jnp.int32),
            pltpu.VMEM((NCH, 128), jnp.int32),
            pltpu.VMEM((NCH, 128), jnp.int32),
            pltpu.VMEM((NCH, 128), jnp.float32),
            pltpu.VMEM((Z_Q,), jnp.float32),
            pltpu.VMEM_SHARED((BUF,), jnp.float32),
            pltpu.VMEM_SHARED((M_LEN,), jnp.float32),
            pltpu.SemaphoreType.DMA,
        ],
    )


def _mm(a, b):
    return lax.dot_general(a, b, (((1,), (0,)), ((), ())),
                           preferred_element_type=jnp.float32)


def _tc_body(cc_m, dd_m, adj_cc, adj_dd,
             ec_w1, ec_b1, ec_w2, ec_b2, ec_w3, ec_b3,
             dc_w1, dc_b1, dc_w2, dc_b2, dc_w3, dc_b3,
             ed_w1, ed_b1, ed_w2, ed_b2, ed_w3, ed_b3,
             sd_w1, sd_b1, sd_w2, sd_b2, sd_w3, sd_b3,
             gc1_w, gc1_b, gc2_w, gc2_b,
             gd1_w, gd1_b, gd2_w, gd2_b,
             wcT, bc, wdT, bd,
             out_ref, cir_ref, dis_ref):
    relu = lambda x: jnp.maximum(x, 0.0)
    sig = lambda x: 1.0 / (1.0 + jnp.exp(-x))

    x_cir = relu(_mm(cc_m[...], ec_w1[...]) + ec_b1[...])
    x_cir = relu(_mm(x_cir, ec_w2[...]) + ec_b2[...])
    x_cir = relu(_mm(x_cir, ec_w3[...]) + ec_b3[...])
    x_cir = relu(_mm(x_cir, dc_w1[...]) + dc_b1[...])
    x_cir = relu(_mm(x_cir, dc_w2[...]) + dc_b2[...])
    x_cir = sig(_mm(x_cir, dc_w3[...]) + dc_b3[...])

    x_dis = relu(_mm(dd_m[...], ed_w1[...]) + ed_b1[...])
    x_dis = relu(_mm(x_dis, ed_w2[...]) + ed_b2[...])
    x_dis = relu(_mm(x_dis, ed_w3[...]) + ed_b3[...])
    x_dis = relu(_mm(x_dis, sd_w1[...]) + sd_b1[...])
    x_dis = relu(_mm(x_dis, sd_w2[...]) + sd_b2[...])
    x_dis = relu(_mm(x_dis, sd_w3[...]) + sd_b3[...])

    def norm_adj(adj, nn):
        rows = lax.broadcasted_iota(jnp.int32, (nn, nn), 0)
        cols = lax.broadcasted_iota(jnp.int32, (nn, nn), 1)
        a = adj[...] + jnp.where(rows == cols, 1.0, 0.0)
        deg = jnp.sum(a, axis=1, keepdims=True)
        dinv = jnp.where(deg > 0, lax.rsqrt(jnp.where(deg > 0, deg, 1.0)), 0.0)
        return a, dinv

    a_cc, dinv_cc = norm_adj(adj_cc, N_CIR)
    a_dd, dinv_dd = norm_adj(adj_dd, N_DIS)

    def gcn(a, dinv, x, w, b):
        h = _mm(x, w[...]) * dinv
        return relu(_mm(a, h) * dinv + b[...])

    f1c = gcn(a_cc, dinv_cc, x_cir, gc1_w, gc1_b)
    f2c = gcn(a_cc, dinv_cc, f1c, gc2_w, gc2_b)
    f1d = gcn(a_dd, dinv_dd, x_dis, gd1_w, gd1_b)
    f2d = gcn(a_dd, dinv_dd, f1d, gd2_w, gd2_b)

    cir = _mm(f1c, wcT[0:128, :]) + _mm(f2c, wcT[128:256, :]) + bc[...]
    dis = _mm(f1d, wdT[0:128, :]) + _mm(f2d, wdT[128:256, :]) + bd[...]

    cir_ref[...] = cir
    dis_ref[...] = dis
    out_ref[...] = lax.dot_general(cir, dis, (((1,), (1,)), ((), ())),
                                   preferred_element_type=jnp.float32)


def kernel(cc_data_matrix, dd_data_matrix, cc_edges, dd_edges,
           ec_w1, ec_b1, ec_w2, ec_b2, ec_w3, ec_b3,
           dc_w1, dc_b1, dc_w2, dc_b2, dc_w3, dc_b3,
           ed_w1, ed_b1, ed_w2, ed_b2, ed_w3, ed_b3,
           sd_w1, sd_b1, sd_w2, sd_b2, sd_w3, sd_b3,
           gc1_w, gc1_b, gc2_w, gc2_b,
           gd1_w, gd1_b, gd2_w, gd2_b,
           cnnc_w, cnnc_b, cnnd_w, cnnd_b):
    i32 = jnp.int32

    def pad(e, n, ne, npadded, nspread):
        npad = npadded - ne
        s = jnp.concatenate([e[0].astype(i32),
                             jnp.arange(npad, dtype=i32) % nspread])
        d = jnp.concatenate([e[1].astype(i32), jnp.full((npad,), n, i32)])
        return s, d

    scc, dcc = pad(cc_edges, N_CIR, E_CC, NS * QE, 111)
    sdd, ddd = pad(dd_edges, N_DIS, E_DD, NS * QE_D, 101)
    src_all = jnp.concatenate([scc, sdd])
    dst_all = jnp.concatenate([dcc, ddd])
    m_cc = cc_data_matrix.reshape(-1)
    m_dd = jnp.concatenate([dd_data_matrix.reshape(-1),
                            jnp.zeros((NS * DD_Q - DD_SZ,), jnp.float32)])
    m_tail = jnp.concatenate([m_cc[CC_SZ - 1:], jnp.zeros((7,), jnp.float32)])
    zeros = jnp.zeros((CC_Q,), jnp.float32)

    adj_flat = _sc_build_adj()(m_cc, m_dd, m_tail, src_all, dst_all, zeros)
    adj_cc = adj_flat[:CC_SZ].reshape(N_CIR, N_CIR)
    adj_dd = adj_flat[DD_OFF:DD_OFF + DD_SZ].reshape(N_DIS, N_DIS)

    biases = [b.reshape(1, -1) for b in
              (ec_b1, ec_b2, ec_b3, dc_b1, dc_b2, dc_b3,
               ed_b1, ed_b2, ed_b3, sd_b1, sd_b2, sd_b3,
               gc1_b, gc2_b, gd1_b, gd2_b)]
    (ec_b1, ec_b2, ec_b3, dc_b1, dc_b2, dc_b3,
     ed_b1, ed_b2, ed_b3, sd_b1, sd_b2, sd_b3,
     gc1_b, gc2_b, gd1_b, gd2_b) = biases
    wcT = cnnc_w.reshape(256, 256).T
    wdT = cnnd_w.reshape(256, 256).T
    bc = cnnc_b.reshape(1, -1)
    bd = cnnd_b.reshape(1, -1)

    out, cir_fea, dis_fea = pl.pallas_call(
        _tc_body,
        out_shape=[
            jax.ShapeDtypeStruct((N_CIR, N_DIS), jnp.float32),
            jax.ShapeDtypeStruct((N_CIR, 256), jnp.float32),
            jax.ShapeDtypeStruct((N_DIS, 256), jnp.float32),
        ],
    )(cc_data_matrix, dd_data_matrix, adj_cc, adj_dd,
      ec_w1, ec_b1, ec_w2, ec_b2, ec_w3, ec_b3,
      dc_w1, dc_b1, dc_w2, dc_b2, dc_w3, dc_b3,
      ed_w1, ed_b1, ed_w2, ed_b2, ed_w3, ed_b3,
      sd_w1, sd_b1, sd_w2, sd_b2, sd_w3, sd_b3,
      gc1_w, gc1_b, gc2_w, gc2_b,
      gd1_w, gd1_b, gd2_w, gd2_b,
      wcT, bc, wdT, bd)
    return out, cir_fea, dis_fea


# speedup vs baseline: 8.8044x; 1.0508x over previous
import functools

import jax
import jax.numpy as jnp
from jax import lax
from jax.experimental import pallas as pl
from jax.experimental.pallas import tpu as pltpu
from jax.experimental.pallas import tpu_sc as plsc

N_CIR, N_DIS = 663, 100
E_CC, E_DD = 10608, 1600
NS = 16
QE = 768
NCH = QE // 128
QE_D = 128
CC_SZ = N_CIR * N_CIR
CC_Q = 27480
CC_QL = 27368
DD_OFF = NS * CC_Q
DD_SZ = N_DIS * N_DIS
DD_Q = 632
BUF = DD_OFF + NS * DD_Q
Z_Q = BUF // NS
GB_D = CC_SZ + 7
M_LEN = GB_D + NS * DD_Q


SBASE = 0
DBASE = NS * QE + NS * QE_D


def _sc_body(m_cc, m_dd, m_tail, e_all, zeros_hbm, out_hbm,
             src_v, dst_v, idxg_v, idxs_v, w_v, stage_v, shared, m_sh, sem):
    c = lax.axis_index("c")
    s = lax.axis_index("s")
    with jax.named_scope("sc_zero"):
        @pl.when(c == 0)
        def _():
            buf = stage_v.at[pl.ds(0, CC_Q)]
            pltpu.sync_copy(zeros_hbm.at[pl.ds(0, CC_Q)], buf)
            pltpu.sync_copy(buf, shared.at[pl.ds(s * CC_Q, CC_Q)])

        @pl.when(c == 1)
        def _():
            buf = stage_v.at[pl.ds(0, DD_Q)]
            pltpu.sync_copy(zeros_hbm.at[pl.ds(0, DD_Q)], buf)
            pltpu.sync_copy(buf, shared.at[pl.ds(DD_OFF + s * DD_Q, DD_Q)])

    with jax.named_scope("sc_mstage"):
        @pl.when((c == 0) & (s < NS - 1))
        def _():
            buf = stage_v.at[pl.ds(0, CC_Q)]
            pltpu.sync_copy(m_cc.at[pl.ds(s * CC_Q, CC_Q)], buf)
            pltpu.sync_copy(buf, m_sh.at[pl.ds(s * CC_Q, CC_Q)])

        @pl.when((c == 0) & (s == NS - 1))
        def _():
            buf = stage_v.at[pl.ds(0, CC_QL)]
            pltpu.sync_copy(m_cc.at[pl.ds((NS - 1) * CC_Q, CC_QL)], buf)
            pltpu.sync_copy(buf, m_sh.at[pl.ds((NS - 1) * CC_Q, CC_QL)])
            tl = stage_v.at[pl.ds(0, 8)]
            pltpu.sync_copy(m_tail, tl)
            pltpu.sync_copy(tl, m_sh.at[pl.ds(CC_SZ - 1, 8)])

        @pl.when(c == 1)
        def _():
            buf = stage_v.at[pl.ds(0, DD_Q)]
            pltpu.sync_copy(m_dd.at[pl.ds(s * DD_Q, DD_Q)], buf)
            pltpu.sync_copy(buf, m_sh.at[pl.ds(GB_D + s * DD_Q, DD_Q)])

    def edge_prep(ebase, nch, n, gbase, sbase):
        qe = nch * 128
        pltpu.sync_copy(e_all.at[pl.ds(SBASE + ebase, qe)],
                        src_v.at[pl.ds(0, qe)])
        pltpu.sync_copy(e_all.at[pl.ds(DBASE + ebase, qe)],
                        dst_v.at[pl.ds(0, qe)])
        for k in range(qe // 16):
            sv = src_v[pl.ds(k * 16, 16)]
            dv = dst_v[pl.ds(k * 16, 16)]
            j, o = k // 8, (k % 8) * 16
            idxg_v[j, pl.ds(o, 16)] = gbase + sv * n + dv
            idxs_v[j, pl.ds(o, 16)] = sbase + dv * n + sv

    with jax.named_scope("sc_idx"):
        @pl.when(c == 0)
        def _():
            edge_prep(s * QE, NCH, N_CIR, 0, 0)

        @pl.when(c == 1)
        def _():
            edge_prep(NS * QE + s * QE_D, 1, N_DIS, GB_D, DD_OFF)

    with jax.named_scope("sc_bar1"):
        plsc.subcore_barrier()

    def gather_scatter(nch):
        copies = [
            pltpu.async_copy(m_sh.at[idxg_v.at[j]], w_v.at[j], sem)
            for j in range(nch)
        ]
        for cp in copies:
            cp.wait()
        for j in range(nch):
            pltpu.sync_copy(w_v.at[j], shared.at[idxs_v.at[j]], add=True)

    with jax.named_scope("sc_gsc"):
        @pl.when(c == 0)
        def _():
            gather_scatter(NCH)

        @pl.when(c == 1)
        def _():
            gather_scatter(1)

    with jax.named_scope("sc_bar2"):
        plsc.subcore_barrier()

    with jax.named_scope("sc_out"):
        @pl.when(c == 0)
        def _():
            buf = stage_v.at[pl.ds(0, CC_Q)]
            pltpu.sync_copy(shared.at[pl.ds(s * CC_Q, CC_Q)], buf)
            pltpu.sync_copy(buf, out_hbm.at[pl.ds(s * CC_Q, CC_Q)])

        @pl.when(c == 1)
        def _():
            off = DD_OFF + s * DD_Q
            buf = stage_v.at[pl.ds(0, DD_Q)]
            pltpu.sync_copy(shared.at[pl.ds(off, DD_Q)], buf)
            pltpu.sync_copy(buf, out_hbm.at[pl.ds(off, DD_Q)])


@functools.cache
def _sc_build_adj():
    return pl.kernel(
        _sc_body,
        out_type=jax.ShapeDtypeStruct((BUF,), jnp.float32),
        mesh=plsc.VectorSubcoreMesh(core_axis_name="c", subcore_axis_name="s"),
        scratch_types=[
            pltpu.VMEM((QE,), jnp.int32),
            pltpu.VMEM((QE,), jnp.int32),
            pltpu.VMEM((NCH, 128), jnp.int32),
            pltpu.VMEM((NCH, 128), jnp.int32),
            pltpu.VMEM((NCH, 128), jnp.float32),
            pltpu.VMEM((Z_Q,), jnp.float32),
            pltpu.VMEM_SHARED((BUF,), jnp.float32),
            pltpu.VMEM_SHARED((M_LEN,), jnp.float32),
            pltpu.SemaphoreType.DMA,
        ],
    )


def _mm(a, b):
    return lax.dot_general(a, b, (((1,), (0,)), ((), ())),
                           preferred_element_type=jnp.float32)


def _tc_mlp_body(cc_m, dd_m,
                 ec_w1, ec_b1, ec_w2, ec_b2, ec_w3, ec_b3,
                 dc_w1, dc_b1, dc_w2, dc_b2, dc_w3, dc_b3,
                 ed_w1, ed_b1, ed_w2, ed_b2, ed_w3, ed_b3,
                 sd_w1, sd_b1, sd_w2, sd_b2, sd_w3, sd_b3,
                 xc_ref, xd_ref):
    relu = lambda x: jnp.maximum(x, 0.0)
    sig = lambda x: 1.0 / (1.0 + jnp.exp(-x))

    x_cir = relu(_mm(cc_m[...], ec_w1[...]) + ec_b1[...])
    x_cir = relu(_mm(x_cir, ec_w2[...]) + ec_b2[...])
    x_cir = relu(_mm(x_cir, ec_w3[...]) + ec_b3[...])
    x_cir = relu(_mm(x_cir, dc_w1[...]) + dc_b1[...])
    x_cir = relu(_mm(x_cir, dc_w2[...]) + dc_b2[...])
    xc_ref[...] = sig(_mm(x_cir, dc_w3[...]) + dc_b3[...])

    x_dis = relu(_mm(dd_m[...], ed_w1[...]) + ed_b1[...])
    x_dis = relu(_mm(x_dis, ed_w2[...]) + ed_b2[...])
    x_dis = relu(_mm(x_dis, ed_w3[...]) + ed_b3[...])
    x_dis = relu(_mm(x_dis, sd_w1[...]) + sd_b1[...])
    x_dis = relu(_mm(x_dis, sd_w2[...]) + sd_b2[...])
    xd_ref[...] = relu(_mm(x_dis, sd_w3[...]) + sd_b3[...])


def _tc_gcn_body(x_cir, x_dis, adj_cc, adj_dd,
                 gc1_w, gc1_b, gc2_w, gc2_b,
                 gd1_w, gd1_b, gd2_w, gd2_b,
                 wc, bc, wd, bd,
                 out_ref, cir_ref, dis_ref):
    relu = lambda x: jnp.maximum(x, 0.0)

    def norm_adj(adj, nn):
        rows = lax.broadcasted_iota(jnp.int32, (nn, nn), 0)
        cols = lax.broadcasted_iota(jnp.int32, (nn, nn), 1)
        a = adj[...] + jnp.where(rows == cols, 1.0, 0.0)
        deg = jnp.sum(a, axis=1, keepdims=True)
        dinv = jnp.where(deg > 0, lax.rsqrt(jnp.where(deg > 0, deg, 1.0)), 0.0)
        return a, dinv

    a_cc, dinv_cc = norm_adj(adj_cc, N_CIR)
    a_dd, dinv_dd = norm_adj(adj_dd, N_DIS)

    def gcn(a, dinv, x, w, b):
        h = _mm(x, w[...]) * dinv
        return relu(_mm(a, h) * dinv + b[...])

    f1c = gcn(a_cc, dinv_cc, x_cir[...], gc1_w, gc1_b)
    f2c = gcn(a_cc, dinv_cc, f1c, gc2_w, gc2_b)
    f1d = gcn(a_dd, dinv_dd, x_dis[...], gd1_w, gd1_b)
    f2d = gcn(a_dd, dinv_dd, f1d, gd2_w, gd2_b)

    def _mmT(a, b):
        return lax.dot_general(a, b, (((1,), (1,)), ((), ())),
                               preferred_element_type=jnp.float32)

    cir = _mmT(f1c, wc[:, 0:128]) + _mmT(f2c, wc[:, 128:256]) + bc[...]
    dis = _mmT(f1d, wd[:, 0:128]) + _mmT(f2d, wd[:, 128:256]) + bd[...]

    cir_ref[...] = cir
    dis_ref[...] = dis
    out_ref[...] = lax.dot_general(cir, dis, (((1,), (1,)), ((), ())),
                                   preferred_element_type=jnp.float32)


def kernel(cc_data_matrix, dd_data_matrix, cc_edges, dd_edges,
           ec_w1, ec_b1, ec_w2, ec_b2, ec_w3, ec_b3,
           dc_w1, dc_b1, dc_w2, dc_b2, dc_w3, dc_b3,
           ed_w1, ed_b1, ed_w2, ed_b2, ed_w3, ed_b3,
           sd_w1, sd_b1, sd_w2, sd_b2, sd_w3, sd_b3,
           gc1_w, gc1_b, gc2_w, gc2_b,
           gd1_w, gd1_b, gd2_w, gd2_b,
           cnnc_w, cnnc_b, cnnd_w, cnnd_b):
    i32 = jnp.int32

    def pad(e, n, ne, npadded, nspread):
        npad = npadded - ne
        s = jnp.concatenate([e[0].astype(i32),
                             jnp.arange(npad, dtype=i32) % nspread])
        d = jnp.concatenate([e[1].astype(i32), jnp.full((npad,), n, i32)])
        return s, d

    scc, dcc = pad(cc_edges, N_CIR, E_CC, NS * QE, 111)
    sdd, ddd = pad(dd_edges, N_DIS, E_DD, NS * QE_D, 101)
    e_all = jnp.concatenate([scc, sdd, dcc, ddd])
    m_cc = cc_data_matrix.reshape(-1)
    m_dd = jnp.concatenate([dd_data_matrix.reshape(-1),
                            jnp.zeros((NS * DD_Q - DD_SZ,), jnp.float32)])
    m_tail = jnp.concatenate([m_cc[CC_SZ - 1:], jnp.zeros((7,), jnp.float32)])
    zeros = jnp.zeros((CC_Q,), jnp.float32)

    adj_flat = _sc_build_adj()(m_cc, m_dd, m_tail, e_all, zeros)
    adj_cc = adj_flat[:CC_SZ].reshape(N_CIR, N_CIR)
    adj_dd = adj_flat[DD_OFF:DD_OFF + DD_SZ].reshape(N_DIS, N_DIS)

    biases = [b.reshape(1, -1) for b in
              (ec_b1, ec_b2, ec_b3, dc_b1, dc_b2, dc_b3,
               ed_b1, ed_b2, ed_b3, sd_b1, sd_b2, sd_b3,
               gc1_b, gc2_b, gd1_b, gd2_b)]
    (ec_b1, ec_b2, ec_b3, dc_b1, dc_b2, dc_b3,
     ed_b1, ed_b2, ed_b3, sd_b1, sd_b2, sd_b3,
     gc1_b, gc2_b, gd1_b, gd2_b) = biases
    wc = cnnc_w.reshape(256, 256)
    wd = cnnd_w.reshape(256, 256)
    bc = cnnc_b.reshape(1, -1)
    bd = cnnd_b.reshape(1, -1)

    x_cir, x_dis = pl.pallas_call(
        _tc_mlp_body,
        out_shape=[
            jax.ShapeDtypeStruct((N_CIR, 64), jnp.float32),
            jax.ShapeDtypeStruct((N_DIS, 64), jnp.float32),
        ],
    )(cc_data_matrix, dd_data_matrix,
      ec_w1, ec_b1, ec_w2, ec_b2, ec_w3, ec_b3,
      dc_w1, dc_b1, dc_w2, dc_b2, dc_w3, dc_b3,
      ed_w1, ed_b1, ed_w2, ed_b2, ed_w3, ed_b3,
      sd_w1, sd_b1, sd_w2, sd_b2, sd_w3, sd_b3)

    out, cir_fea, dis_fea = pl.pallas_call(
        _tc_gcn_body,
        out_shape=[
            jax.ShapeDtypeStruct((N_CIR, N_DIS), jnp.float32),
            jax.ShapeDtypeStruct((N_CIR, 256), jnp.float32),
            jax.ShapeDtypeStruct((N_DIS, 256), jnp.float32),
        ],
    )(x_cir, x_dis, adj_cc, adj_dd,
      gc1_w, gc1_b, gc2_w, gc2_b,
      gd1_w, gd1_b, gd2_w, gd2_b,
      wc, bc, wd, bd)
    return out, cir_fea, dis_fea


# speedup vs baseline: 9.5155x; 1.0808x over previous
import functools

import jax
import jax.numpy as jnp
from jax import lax
from jax.experimental import pallas as pl
from jax.experimental.pallas import tpu as pltpu
from jax.experimental.pallas import tpu_sc as plsc

N_CIR, N_DIS = 663, 100
E_CC, E_DD = 10608, 1600
NS = 16
QE = 768
NCH = QE // 128
QE_D = 128
CC_SZ = N_CIR * N_CIR
CC_Q = 27480
CC_QL = 27368
DD_OFF = NS * CC_Q
DD_SZ = N_DIS * N_DIS
DD_Q = 632
BUF = DD_OFF + NS * DD_Q
Z_Q = BUF // NS
GB_D = CC_SZ + 7
M_LEN = GB_D + NS * DD_Q


SBASE = 0
DBASE = NS * QE + NS * QE_D
CC_QS = 27368
MT_OFF = NS * CC_QS
MT_LEN = GB_D - MT_OFF
AUX_Z = NS * DD_Q


def _sc_body(m_cc, m_tail, aux, e_all, out_hbm,
             src_v, dst_v, idxg_v, idxs_v, w_v, stage_v, zbuf_v,
             shared, m_sh, semE, semZ, semM, semG, semS):
    c = lax.axis_index("c")
    s = lax.axis_index("s")

    def part_a(qe, nch, n, gbase, sbase, ebase, zq, zoff, m_hbm, moff, mq,
               m_sh_off):
        es = src_v.at[pl.ds(0, qe)]
        ed = dst_v.at[pl.ds(0, qe)]
        he1 = pltpu.async_copy(e_all.at[pl.ds(SBASE + ebase, qe)], es, semE)
        he2 = pltpu.async_copy(e_all.at[pl.ds(DBASE + ebase, qe)], ed, semE)
        zb = zbuf_v.at[pl.ds(0, zq)]
        hz = pltpu.async_copy(aux.at[pl.ds(AUX_Z, zq)], zb, semZ)
        mb = stage_v.at[pl.ds(0, mq)]
        hm = pltpu.async_copy(m_hbm.at[pl.ds(moff, mq)], mb, semM)
        he1.wait()
        he2.wait()
        for k in range(qe // 16):
            sv = src_v[pl.ds(k * 16, 16)]
            dv = dst_v[pl.ds(k * 16, 16)]
            j, o = k // 8, (k % 8) * 16
            idxg_v[j, pl.ds(o, 16)] = gbase + sv * n + dv
            idxs_v[j, pl.ds(o, 16)] = sbase + dv * n + sv
        hz.wait()
        pltpu.async_copy(zb, shared.at[pl.ds(zoff, zq)], semZ)
        hm.wait()
        pltpu.async_copy(mb, m_sh.at[pl.ds(m_sh_off, mq)], semM)
        pltpu.make_async_copy(zb, shared.at[pl.ds(zoff, zq)], semZ).wait()
        pltpu.make_async_copy(mb, m_sh.at[pl.ds(m_sh_off, mq)], semM).wait()

    with jax.named_scope("sc_pre"):
        @pl.when(c == 0)
        def _():
            part_a(QE, NCH, N_CIR, 0, 0, s * QE,
                   CC_Q, s * CC_Q, m_cc, s * CC_QS, CC_QS, s * CC_QS)

            @pl.when(s == NS - 1)
            def _():
                tl = zbuf_v.at[pl.ds(0, MT_LEN)]
                pltpu.sync_copy(m_tail, tl)
                pltpu.sync_copy(tl, m_sh.at[pl.ds(MT_OFF, MT_LEN)])

        @pl.when(c == 1)
        def _():
            part_a(QE_D, 1, N_DIS, GB_D, DD_OFF, NS * QE + s * QE_D,
                   DD_Q, DD_OFF + s * DD_Q, aux, s * DD_Q, DD_Q,
                   GB_D + s * DD_Q)

    with jax.named_scope("sc_bar1"):
        plsc.subcore_barrier()

    def part_b(nch):
        gs = [pltpu.async_copy(m_sh.at[idxg_v.at[j]], w_v.at[j], semG)
              for j in range(nch)]
        for h in gs:
            h.wait()
        ss = [pltpu.async_copy(w_v.at[j], shared.at[idxs_v.at[j]], semS,
                               add=True)
              for j in range(nch)]
        for h in ss:
            h.wait()

    with jax.named_scope("sc_gsc"):
        @pl.when(c == 0)
        def _():
            part_b(NCH)

        @pl.when(c == 1)
        def _():
            part_b(1)

    with jax.named_scope("sc_bar2"):
        plsc.subcore_barrier()

    with jax.named_scope("sc_out"):
        @pl.when(c == 0)
        def _():
            buf = stage_v.at[pl.ds(0, CC_Q)]
            pltpu.sync_copy(shared.at[pl.ds(s * CC_Q, CC_Q)], buf)
            pltpu.sync_copy(buf, out_hbm.at[pl.ds(s * CC_Q, CC_Q)])

        @pl.when(c == 1)
        def _():
            off = DD_OFF + s * DD_Q
            buf = stage_v.at[pl.ds(0, DD_Q)]
            pltpu.sync_copy(shared.at[pl.ds(off, DD_Q)], buf)
            pltpu.sync_copy(buf, out_hbm.at[pl.ds(off, DD_Q)])


@functools.cache
def _sc_build_adj():
    return pl.kernel(
        _sc_body,
        out_type=jax.ShapeDtypeStruct((BUF,), jnp.float32),
        mesh=plsc.VectorSubcoreMesh(core_axis_name="c", subcore_axis_name="s"),
        scratch_types=[
            pltpu.VMEM((QE,), jnp.int32),
            pltpu.VMEM((QE,), jnp.int32),
            pltpu.VMEM((NCH, 128), jnp.int32),
            pltpu.VMEM((NCH, 128), jnp.int32),
            pltpu.VMEM((NCH, 128), jnp.float32),
            pltpu.VMEM((CC_Q,), jnp.float32),
            pltpu.VMEM((CC_Q,), jnp.float32),
            pltpu.VMEM_SHARED((BUF,), jnp.float32),
            pltpu.VMEM_SHARED((M_LEN,), jnp.float32),
            pltpu.SemaphoreType.DMA,
            pltpu.SemaphoreType.DMA,
            pltpu.SemaphoreType.DMA,
            pltpu.SemaphoreType.DMA,
            pltpu.SemaphoreType.DMA,
        ],
    )


def _mm(a, b):
    return lax.dot_general(a, b, (((1,), (0,)), ((), ())),
                           preferred_element_type=jnp.float32)


def _tc_mlp_body(cc_m, dd_m,
                 ec_w1, ec_b1, ec_w2, ec_b2, ec_w3, ec_b3,
                 dc_w1, dc_b1, dc_w2, dc_b2, dc_w3, dc_b3,
                 ed_w1, ed_b1, ed_w2, ed_b2, ed_w3, ed_b3,
                 sd_w1, sd_b1, sd_w2, sd_b2, sd_w3, sd_b3,
                 xc_ref, xd_ref):
    relu = lambda x: jnp.maximum(x, 0.0)
    sig = lambda x: 1.0 / (1.0 + jnp.exp(-x))

    x_cir = relu(_mm(cc_m[...], ec_w1[...]) + ec_b1[...])
    x_cir = relu(_mm(x_cir, ec_w2[...]) + ec_b2[...])
    x_cir = relu(_mm(x_cir, ec_w3[...]) + ec_b3[...])
    x_cir = relu(_mm(x_cir, dc_w1[...]) + dc_b1[...])
    x_cir = relu(_mm(x_cir, dc_w2[...]) + dc_b2[...])
    xc_ref[...] = sig(_mm(x_cir, dc_w3[...]) + dc_b3[...])

    x_dis = relu(_mm(dd_m[...], ed_w1[...]) + ed_b1[...])
    x_dis = relu(_mm(x_dis, ed_w2[...]) + ed_b2[...])
    x_dis = relu(_mm(x_dis, ed_w3[...]) + ed_b3[...])
    x_dis = relu(_mm(x_dis, sd_w1[...]) + sd_b1[...])
    x_dis = relu(_mm(x_dis, sd_w2[...]) + sd_b2[...])
    xd_ref[...] = relu(_mm(x_dis, sd_w3[...]) + sd_b3[...])


def _tc_gcn_body(x_cir, x_dis, adj_cc, adj_dd,
                 gc1_w, gc1_b, gc2_w, gc2_b,
                 gd1_w, gd1_b, gd2_w, gd2_b,
                 wc, bc, wd, bd,
                 out_ref, cir_ref, dis_ref):
    relu = lambda x: jnp.maximum(x, 0.0)

    def norm_adj(adj, nn):
        rows = lax.broadcasted_iota(jnp.int32, (nn, nn), 0)
        cols = lax.broadcasted_iota(jnp.int32, (nn, nn), 1)
        a = adj[...] + jnp.where(rows == cols, 1.0, 0.0)
        deg = jnp.sum(a, axis=1, keepdims=True)
        dinv = jnp.where(deg > 0, lax.rsqrt(jnp.where(deg > 0, deg, 1.0)), 0.0)
        return a, dinv

    a_cc, dinv_cc = norm_adj(adj_cc, N_CIR)
    a_dd, dinv_dd = norm_adj(adj_dd, N_DIS)

    def gcn(a, dinv, x, w, b):
        h = _mm(x, w[...]) * dinv
        return relu(_mm(a, h) * dinv + b[...])

    f1c = gcn(a_cc, dinv_cc, x_cir[...], gc1_w, gc1_b)
    f2c = gcn(a_cc, dinv_cc, f1c, gc2_w, gc2_b)
    f1d = gcn(a_dd, dinv_dd, x_dis[...], gd1_w, gd1_b)
    f2d = gcn(a_dd, dinv_dd, f1d, gd2_w, gd2_b)

    def _mmT(a, b):
        return lax.dot_general(a, b, (((1,), (1,)), ((), ())),
                               preferred_element_type=jnp.float32)

    cir = _mmT(f1c, wc[:, 0:128]) + _mmT(f2c, wc[:, 128:256]) + bc[...]
    dis = _mmT(f1d, wd[:, 0:128]) + _mmT(f2d, wd[:, 128:256]) + bd[...]

    cir_ref[...] = cir
    dis_ref[...] = dis
    out_ref[...] = lax.dot_general(cir, dis, (((1,), (1,)), ((), ())),
                                   preferred_element_type=jnp.float32)


def kernel(cc_data_matrix, dd_data_matrix, cc_edges, dd_edges,
           ec_w1, ec_b1, ec_w2, ec_b2, ec_w3, ec_b3,
           dc_w1, dc_b1, dc_w2, dc_b2, dc_w3, dc_b3,
           ed_w1, ed_b1, ed_w2, ed_b2, ed_w3, ed_b3,
           sd_w1, sd_b1, sd_w2, sd_b2, sd_w3, sd_b3,
           gc1_w, gc1_b, gc2_w, gc2_b,
           gd1_w, gd1_b, gd2_w, gd2_b,
           cnnc_w, cnnc_b, cnnd_w, cnnd_b):
    i32 = jnp.int32

    i = jnp.arange(NS * QE - E_CC, dtype=i32)
    j = jnp.arange(NS * QE_D - E_DD, dtype=i32)
    e_all = jnp.concatenate([
        cc_edges[0].astype(i32), i % 111, dd_edges[0].astype(i32), j % 101,
        cc_edges[1].astype(i32), jnp.full(i.shape, N_CIR, i32),
        dd_edges[1].astype(i32), jnp.full(j.shape, N_DIS, i32)])
    m_cc = cc_data_matrix.reshape(-1)
    aux = jnp.concatenate([dd_data_matrix.reshape(-1),
                           jnp.zeros((112 + CC_Q,), jnp.float32)])
    m_tail = jnp.concatenate([m_cc[MT_OFF:], jnp.zeros((7,), jnp.float32)])

    adj_flat = _sc_build_adj()(m_cc, m_tail, aux, e_all)
    adj_cc = adj_flat[:CC_SZ].reshape(N_CIR, N_CIR)
    adj_dd = adj_flat[DD_OFF:DD_OFF + DD_SZ].reshape(N_DIS, N_DIS)

    biases = [b.reshape(1, -1) for b in
              (ec_b1, ec_b2, ec_b3, dc_b1, dc_b2, dc_b3,
               ed_b1, ed_b2, ed_b3, sd_b1, sd_b2, sd_b3,
               gc1_b, gc2_b, gd1_b, gd2_b)]
    (ec_b1, ec_b2, ec_b3, dc_b1, dc_b2, dc_b3,
     ed_b1, ed_b2, ed_b3, sd_b1, sd_b2, sd_b3,
     gc1_b, gc2_b, gd1_b, gd2_b) = biases
    wc = cnnc_w.reshape(256, 256)
    wd = cnnd_w.reshape(256, 256)
    bc = cnnc_b.reshape(1, -1)
    bd = cnnd_b.reshape(1, -1)

    x_cir, x_dis = pl.pallas_call(
        _tc_mlp_body,
        out_shape=[
            jax.ShapeDtypeStruct((N_CIR, 64), jnp.float32),
            jax.ShapeDtypeStruct((N_DIS, 64), jnp.float32),
        ],
    )(cc_data_matrix, dd_data_matrix,
      ec_w1, ec_b1, ec_w2, ec_b2, ec_w3, ec_b3,
      dc_w1, dc_b1, dc_w2, dc_b2, dc_w3, dc_b3,
      ed_w1, ed_b1, ed_w2, ed_b2, ed_w3, ed_b3,
      sd_w1, sd_b1, sd_w2, sd_b2, sd_w3, sd_b3)

    out, cir_fea, dis_fea = pl.pallas_call(
        _tc_gcn_body,
        out_shape=[
            jax.ShapeDtypeStruct((N_CIR, N_DIS), jnp.float32),
            jax.ShapeDtypeStruct((N_CIR, 256), jnp.float32),
            jax.ShapeDtypeStruct((N_DIS, 256), jnp.float32),
        ],
    )(x_cir, x_dis, adj_cc, adj_dd,
      gc1_w, gc1_b, gc2_w, gc2_b,
      gd1_w, gd1_b, gd2_w, gd2_b,
      wc, bc, wd, bd)
    return out, cir_fea, dis_fea


# speedup vs baseline: 10.1563x; 1.0673x over previous
import functools

import jax
import jax.numpy as jnp
from jax import lax
from jax.experimental import pallas as pl
from jax.experimental.pallas import tpu as pltpu
from jax.experimental.pallas import tpu_sc as plsc

N_CIR, N_DIS = 663, 100
E_CC, E_DD = 10608, 1600
NS = 16
QE = 768
NCH = QE // 128
QE_D = 128
CC_SZ = N_CIR * N_CIR
CC_Q = 27480
CC_QL = 27368
DD_OFF = NS * CC_Q
DD_SZ = N_DIS * N_DIS
DD_Q = 632
BUF = DD_OFF + NS * DD_Q
Z_Q = BUF // NS
GB_D = CC_SZ + 7
M_LEN = GB_D + NS * DD_Q


SBASE = 0
DBASE = NS * QE + NS * QE_D
CC_QS = 27368
MT_OFF = NS * CC_QS
MT_LEN = GB_D - MT_OFF
AUX_Z = NS * DD_Q


def _sc_body(m_cc, m_tail, aux, e_all, out_hbm,
             src_v, dst_v, idxg_v, idxs_v, w_v, stage_v, zbuf_v,
             shared, m_sh, semE, semZ, semM, semG, semS):
    c = lax.axis_index("c")
    s = lax.axis_index("s")

    def part_a(qe, nch, n, gbase, sbase, ebase, zq, zoff, m_hbm, moff, mq,
               m_sh_off):
        es = src_v.at[pl.ds(0, qe)]
        ed = dst_v.at[pl.ds(0, qe)]
        he1 = pltpu.async_copy(e_all.at[pl.ds(SBASE + ebase, qe)], es, semE)
        he2 = pltpu.async_copy(e_all.at[pl.ds(DBASE + ebase, qe)], ed, semE)
        mb = stage_v.at[pl.ds(0, mq)]
        hm = pltpu.async_copy(m_hbm.at[pl.ds(moff, mq)], mb, semM)
        zero16 = jnp.zeros((16,), jnp.float32)

        def zbody(i, carry):
            for k in range(16):
                zbuf_v[pl.ds(i * 256 + k * 16, 16)] = zero16
            return carry

        lax.fori_loop(0, (zq + 255) // 256, zbody, 0)
        zb = zbuf_v.at[pl.ds(0, zq)]
        hz = pltpu.async_copy(zb, shared.at[pl.ds(zoff, zq)], semZ)
        he1.wait()
        he2.wait()
        for k in range(qe // 16):
            sv = src_v[pl.ds(k * 16, 16)]
            dv = dst_v[pl.ds(k * 16, 16)]
            j, o = k // 8, (k % 8) * 16
            idxg_v[j, pl.ds(o, 16)] = gbase + sv * n + dv
            idxs_v[j, pl.ds(o, 16)] = sbase + dv * n + sv
        hm.wait()
        hms = pltpu.async_copy(mb, m_sh.at[pl.ds(m_sh_off, mq)], semM)
        hz.wait()
        hms.wait()

    with jax.named_scope("sc_pre"):
        @pl.when(c == 0)
        def _():
            part_a(QE, NCH, N_CIR, 0, 0, s * QE,
                   CC_Q, s * CC_Q, m_cc, s * CC_QS, CC_QS, s * CC_QS)

            @pl.when(s == NS - 1)
            def _():
                tl = zbuf_v.at[pl.ds(0, MT_LEN)]
                pltpu.sync_copy(m_tail, tl)
                pltpu.sync_copy(tl, m_sh.at[pl.ds(MT_OFF, MT_LEN)])

        @pl.when(c == 1)
        def _():
            part_a(QE_D, 1, N_DIS, GB_D, DD_OFF, NS * QE + s * QE_D,
                   DD_Q, DD_OFF + s * DD_Q, aux, s * DD_Q, DD_Q,
                   GB_D + s * DD_Q)

    with jax.named_scope("sc_bar1"):
        plsc.subcore_barrier()

    def part_b(nch):
        gs = [pltpu.async_copy(m_sh.at[idxg_v.at[j]], w_v.at[j], semG)
              for j in range(nch)]
        for h in gs:
            h.wait()
        ss = [pltpu.async_copy(w_v.at[j], shared.at[idxs_v.at[j]], semS,
                               add=True)
              for j in range(nch)]
        for h in ss:
            h.wait()

    with jax.named_scope("sc_gsc"):
        @pl.when(c == 0)
        def _():
            part_b(NCH)

        @pl.when(c == 1)
        def _():
            part_b(1)

    with jax.named_scope("sc_bar2"):
        plsc.subcore_barrier()

    with jax.named_scope("sc_out"):
        @pl.when(c == 0)
        def _():
            ha, hb = 13744, CC_Q - 13744
            bufa = stage_v.at[pl.ds(0, ha)]
            bufb = stage_v.at[pl.ds(ha, hb)]
            pltpu.sync_copy(shared.at[pl.ds(s * CC_Q, ha)], bufa)
            h = pltpu.async_copy(bufa, out_hbm.at[pl.ds(s * CC_Q, ha)], semM)
            pltpu.sync_copy(shared.at[pl.ds(s * CC_Q + ha, hb)], bufb)
            h.wait()
            pltpu.sync_copy(bufb, out_hbm.at[pl.ds(s * CC_Q + ha, hb)])

        @pl.when(c == 1)
        def _():
            off = DD_OFF + s * DD_Q
            buf = stage_v.at[pl.ds(0, DD_Q)]
            pltpu.sync_copy(shared.at[pl.ds(off, DD_Q)], buf)
            pltpu.sync_copy(buf, out_hbm.at[pl.ds(off, DD_Q)])


@functools.cache
def _sc_build_adj():
    return pl.kernel(
        _sc_body,
        out_type=jax.ShapeDtypeStruct((BUF,), jnp.float32),
        mesh=plsc.VectorSubcoreMesh(core_axis_name="c", subcore_axis_name="s"),
        scratch_types=[
            pltpu.VMEM((QE,), jnp.int32),
            pltpu.VMEM((QE,), jnp.int32),
            pltpu.VMEM((NCH, 128), jnp.int32),
            pltpu.VMEM((NCH, 128), jnp.int32),
            pltpu.VMEM((NCH, 128), jnp.float32),
            pltpu.VMEM((CC_Q,), jnp.float32),
            pltpu.VMEM((27648,), jnp.float32),
            pltpu.VMEM_SHARED((BUF,), jnp.float32),
            pltpu.VMEM_SHARED((M_LEN,), jnp.float32),
            pltpu.SemaphoreType.DMA,
            pltpu.SemaphoreType.DMA,
            pltpu.SemaphoreType.DMA,
            pltpu.SemaphoreType.DMA,
            pltpu.SemaphoreType.DMA,
        ],
    )


def _mm(a, b):
    return lax.dot_general(a, b, (((1,), (0,)), ((), ())),
                           preferred_element_type=jnp.float32)


def _tc_mlp_body(cc_m, dd_m,
                 ec_w1, ec_b1, ec_w2, ec_b2, ec_w3, ec_b3,
                 dc_w1, dc_b1, dc_w2, dc_b2, dc_w3, dc_b3,
                 ed_w1, ed_b1, ed_w2, ed_b2, ed_w3, ed_b3,
                 sd_w1, sd_b1, sd_w2, sd_b2, sd_w3, sd_b3,
                 xc_ref, xd_ref):
    relu = lambda x: jnp.maximum(x, 0.0)
    sig = lambda x: 1.0 / (1.0 + jnp.exp(-x))

    x_cir = relu(_mm(cc_m[...], ec_w1[...]) + ec_b1[...])
    x_cir = relu(_mm(x_cir, ec_w2[...]) + ec_b2[...])
    x_cir = relu(_mm(x_cir, ec_w3[...]) + ec_b3[...])
    x_cir = relu(_mm(x_cir, dc_w1[...]) + dc_b1[...])
    x_cir = relu(_mm(x_cir, dc_w2[...]) + dc_b2[...])
    xc_ref[...] = sig(_mm(x_cir, dc_w3[...]) + dc_b3[...])

    x_dis = relu(_mm(dd_m[...], ed_w1[...]) + ed_b1[...])
    x_dis = relu(_mm(x_dis, ed_w2[...]) + ed_b2[...])
    x_dis = relu(_mm(x_dis, ed_w3[...]) + ed_b3[...])
    x_dis = relu(_mm(x_dis, sd_w1[...]) + sd_b1[...])
    x_dis = relu(_mm(x_dis, sd_w2[...]) + sd_b2[...])
    xd_ref[...] = relu(_mm(x_dis, sd_w3[...]) + sd_b3[...])


def _tc_gcn_body(x_cir, x_dis, adj_cc, adj_dd,
                 gc1_w, gc1_b, gc2_w, gc2_b,
                 gd1_w, gd1_b, gd2_w, gd2_b,
                 wc, bc, wd, bd,
                 out_ref, cir_ref, dis_ref):
    relu = lambda x: jnp.maximum(x, 0.0)

    def norm_adj(adj, nn):
        rows = lax.broadcasted_iota(jnp.int32, (nn, nn), 0)
        cols = lax.broadcasted_iota(jnp.int32, (nn, nn), 1)
        a = adj[...] + jnp.where(rows == cols, 1.0, 0.0)
        deg = jnp.sum(a, axis=1, keepdims=True)
        dinv = jnp.where(deg > 0, lax.rsqrt(jnp.where(deg > 0, deg, 1.0)), 0.0)
        return a, dinv

    a_cc, dinv_cc = norm_adj(adj_cc, N_CIR)
    a_dd, dinv_dd = norm_adj(adj_dd, N_DIS)

    def gcn(a, dinv, x, w, b):
        h = _mm(x, w[...]) * dinv
        return relu(_mm(a, h) * dinv + b[...])

    f1c = gcn(a_cc, dinv_cc, x_cir[...], gc1_w, gc1_b)
    f2c = gcn(a_cc, dinv_cc, f1c, gc2_w, gc2_b)
    f1d = gcn(a_dd, dinv_dd, x_dis[...], gd1_w, gd1_b)
    f2d = gcn(a_dd, dinv_dd, f1d, gd2_w, gd2_b)

    def _mmT(a, b):
        return lax.dot_general(a, b, (((1,), (1,)), ((), ())),
                               preferred_element_type=jnp.float32)

    cir = _mmT(f1c, wc[:, 0:128]) + _mmT(f2c, wc[:, 128:256]) + bc[...]
    dis = _mmT(f1d, wd[:, 0:128]) + _mmT(f2d, wd[:, 128:256]) + bd[...]

    cir_ref[...] = cir
    dis_ref[...] = dis
    out_ref[...] = lax.dot_general(cir, dis, (((1,), (1,)), ((), ())),
                                   preferred_element_type=jnp.float32)


def kernel(cc_data_matrix, dd_data_matrix, cc_edges, dd_edges,
           ec_w1, ec_b1, ec_w2, ec_b2, ec_w3, ec_b3,
           dc_w1, dc_b1, dc_w2, dc_b2, dc_w3, dc_b3,
           ed_w1, ed_b1, ed_w2, ed_b2, ed_w3, ed_b3,
           sd_w1, sd_b1, sd_w2, sd_b2, sd_w3, sd_b3,
           gc1_w, gc1_b, gc2_w, gc2_b,
           gd1_w, gd1_b, gd2_w, gd2_b,
           cnnc_w, cnnc_b, cnnd_w, cnnd_b):
    i32 = jnp.int32

    i = jnp.arange(NS * QE - E_CC, dtype=i32)
    j = jnp.arange(NS * QE_D - E_DD, dtype=i32)
    e_all = jnp.concatenate([
        cc_edges[0].astype(i32), i % 111, dd_edges[0].astype(i32), j % 101,
        cc_edges[1].astype(i32), jnp.full(i.shape, N_CIR, i32),
        dd_edges[1].astype(i32), jnp.full(j.shape, N_DIS, i32)])
    m_cc = cc_data_matrix.reshape(-1)
    aux = jnp.concatenate([dd_data_matrix.reshape(-1),
                           jnp.zeros((112,), jnp.float32)])
    m_tail = jnp.concatenate([m_cc[MT_OFF:], jnp.zeros((7,), jnp.float32)])

    adj_flat = _sc_build_adj()(m_cc, m_tail, aux, e_all)
    adj_cc = adj_flat[:CC_SZ].reshape(N_CIR, N_CIR)
    adj_dd = adj_flat[DD_OFF:DD_OFF + DD_SZ].reshape(N_DIS, N_DIS)

    biases = [b.reshape(1, -1) for b in
              (ec_b1, ec_b2, ec_b3, dc_b1, dc_b2, dc_b3,
               ed_b1, ed_b2, ed_b3, sd_b1, sd_b2, sd_b3,
               gc1_b, gc2_b, gd1_b, gd2_b)]
    (ec_b1, ec_b2, ec_b3, dc_b1, dc_b2, dc_b3,
     ed_b1, ed_b2, ed_b3, sd_b1, sd_b2, sd_b3,
     gc1_b, gc2_b, gd1_b, gd2_b) = biases
    wc = cnnc_w.reshape(256, 256)
    wd = cnnd_w.reshape(256, 256)
    bc = cnnc_b.reshape(1, -1)
    bd = cnnd_b.reshape(1, -1)

    x_cir, x_dis = pl.pallas_call(
        _tc_mlp_body,
        out_shape=[
            jax.ShapeDtypeStruct((N_CIR, 64), jnp.float32),
            jax.ShapeDtypeStruct((N_DIS, 64), jnp.float32),
        ],
    )(cc_data_matrix, dd_data_matrix,
      ec_w1, ec_b1, ec_w2, ec_b2, ec_w3, ec_b3,
      dc_w1, dc_b1, dc_w2, dc_b2, dc_w3, dc_b3,
      ed_w1, ed_b1, ed_w2, ed_b2, ed_w3, ed_b3,
      sd_w1, sd_b1, sd_w2, sd_b2, sd_w3, sd_b3)

    out, cir_fea, dis_fea = pl.pallas_call(
        _tc_gcn_body,
        out_shape=[
            jax.ShapeDtypeStruct((N_CIR, N_DIS), jnp.float32),
            jax.ShapeDtypeStruct((N_CIR, 256), jnp.float32),
            jax.ShapeDtypeStruct((N_DIS, 256), jnp.float32),
        ],
    )(x_cir, x_dis, adj_cc, adj_dd,
      gc1_w, gc1_b, gc2_w, gc2_b,
      gd1_w, gd1_b, gd2_w, gd2_b,
      wc, bc, wd, bd)
    return out, cir_fea, dis_fea


# speedup vs baseline: 10.3062x; 1.0148x over previous
import functools

import jax
import jax.numpy as jnp
from jax import lax
from jax.experimental import pallas as pl
from jax.experimental.pallas import tpu as pltpu
from jax.experimental.pallas import tpu_sc as plsc

N_CIR, N_DIS = 663, 100
E_CC, E_DD = 10608, 1600
NS = 16
QE = 768
NCH = QE // 128
QE_D = 128
CC_SZ = N_CIR * N_CIR
CC_Q = 27480
CC_QL = 27368
DD_OFF = NS * CC_Q
DD_SZ = N_DIS * N_DIS
DD_Q = 632
BUF = DD_OFF + NS * DD_Q
Z_Q = BUF // NS
GB_D = CC_SZ + 7
M_LEN = GB_D + NS * DD_Q


SBASE = 0
DBASE = NS * QE + NS * QE_D
CC_QS = 27368
MT_OFF = NS * CC_QS
MT_LEN = GB_D - MT_OFF
AUX_Z = NS * DD_Q


def _sc_body(m_cc, m_tail, aux, e_all, out_cc, out_dd,
             src_v, dst_v, idxg_v, idxs_v, w_v, stage_v, zbuf_v,
             shared, m_sh, semE, semZ, semM, semG, semS):
    c = lax.axis_index("c")
    s = lax.axis_index("s")

    def part_a(qe, nch, n, gbase, sbase, ebase, zq, zoff, m_hbm, moff, mq,
               m_sh_off):
        es = src_v.at[pl.ds(0, qe)]
        ed = dst_v.at[pl.ds(0, qe)]
        he1 = pltpu.async_copy(e_all.at[pl.ds(SBASE + ebase, qe)], es, semE)
        he2 = pltpu.async_copy(e_all.at[pl.ds(DBASE + ebase, qe)], ed, semE)
        mb = stage_v.at[pl.ds(0, mq)]
        hm = pltpu.async_copy(m_hbm.at[pl.ds(moff, mq)], mb, semM)
        zero16 = jnp.zeros((16,), jnp.float32)

        def zbody(i, carry):
            for k in range(16):
                zbuf_v[pl.ds(i * 256 + k * 16, 16)] = zero16
            return carry

        lax.fori_loop(0, (zq + 255) // 256, zbody, 0)
        zb = zbuf_v.at[pl.ds(0, zq)]
        hz = pltpu.async_copy(zb, shared.at[pl.ds(zoff, zq)], semZ)
        he1.wait()
        he2.wait()
        for k in range(qe // 16):
            sv = src_v[pl.ds(k * 16, 16)]
            dv = dst_v[pl.ds(k * 16, 16)]
            j, o = k // 8, (k % 8) * 16
            idxg_v[j, pl.ds(o, 16)] = gbase + sv * n + dv
            idxs_v[j, pl.ds(o, 16)] = sbase + dv * n + sv
        hm.wait()
        hms = pltpu.async_copy(mb, m_sh.at[pl.ds(m_sh_off, mq)], semM)
        hz.wait()
        hms.wait()

    with jax.named_scope("sc_pre"):
        @pl.when(c == 0)
        def _():
            part_a(QE, NCH, N_CIR, 0, 0, s * QE,
                   CC_Q, s * CC_Q, m_cc, s * CC_QS, CC_QS, s * CC_QS)

            @pl.when(s == NS - 1)
            def _():
                tl = zbuf_v.at[pl.ds(0, MT_LEN)]
                pltpu.sync_copy(m_tail, tl)
                pltpu.sync_copy(tl, m_sh.at[pl.ds(MT_OFF, MT_LEN)])

        @pl.when(c == 1)
        def _():
            part_a(QE_D, 1, N_DIS, GB_D, DD_OFF, NS * QE + s * QE_D,
                   DD_Q, DD_OFF + s * DD_Q, aux, s * DD_Q, DD_Q,
                   GB_D + s * DD_Q)

    with jax.named_scope("sc_bar1"):
        plsc.subcore_barrier()

    def part_b(nch):
        gs = [pltpu.async_copy(m_sh.at[idxg_v.at[j]], w_v.at[j], semG)
              for j in range(nch)]
        for h in gs:
            h.wait()
        ss = [pltpu.async_copy(w_v.at[j], shared.at[idxs_v.at[j]], semS,
                               add=True)
              for j in range(nch)]
        for h in ss:
            h.wait()

    with jax.named_scope("sc_gsc"):
        @pl.when(c == 0)
        def _():
            part_b(NCH)

        @pl.when(c == 1)
        def _():
            part_b(1)

    with jax.named_scope("sc_bar2"):
        plsc.subcore_barrier()

    with jax.named_scope("sc_out"):
        @pl.when(c == 0)
        def _():
            ha, hb = 13744, CC_Q - 13744
            bufa = stage_v.at[pl.ds(0, ha)]
            bufb = stage_v.at[pl.ds(ha, hb)]
            pltpu.sync_copy(shared.at[pl.ds(s * CC_Q, ha)], bufa)
            h = pltpu.async_copy(bufa, out_cc.at[pl.ds(s * CC_Q, ha)], semM)
            pltpu.sync_copy(shared.at[pl.ds(s * CC_Q + ha, hb)], bufb)
            h.wait()
            pltpu.sync_copy(bufb, out_cc.at[pl.ds(s * CC_Q + ha, hb)])

        @pl.when(c == 1)
        def _():
            buf = stage_v.at[pl.ds(0, DD_Q)]
            pltpu.sync_copy(shared.at[pl.ds(DD_OFF + s * DD_Q, DD_Q)], buf)
            pltpu.sync_copy(buf, out_dd.at[pl.ds(s * DD_Q, DD_Q)])


@functools.cache
def _sc_build_adj():
    return pl.kernel(
        _sc_body,
        out_type=[
            jax.ShapeDtypeStruct((NS * CC_Q,), jnp.float32),
            jax.ShapeDtypeStruct((NS * DD_Q,), jnp.float32),
        ],
        mesh=plsc.VectorSubcoreMesh(core_axis_name="c", subcore_axis_name="s"),
        scratch_types=[
            pltpu.VMEM((QE,), jnp.int32),
            pltpu.VMEM((QE,), jnp.int32),
            pltpu.VMEM((NCH, 128), jnp.int32),
            pltpu.VMEM((NCH, 128), jnp.int32),
            pltpu.VMEM((NCH, 128), jnp.float32),
            pltpu.VMEM((CC_Q,), jnp.float32),
            pltpu.VMEM((27648,), jnp.float32),
            pltpu.VMEM_SHARED((BUF,), jnp.float32),
            pltpu.VMEM_SHARED((M_LEN,), jnp.float32),
            pltpu.SemaphoreType.DMA,
            pltpu.SemaphoreType.DMA,
            pltpu.SemaphoreType.DMA,
            pltpu.SemaphoreType.DMA,
            pltpu.SemaphoreType.DMA,
        ],
    )


def _mm(a, b):
    return lax.dot_general(a, b, (((1,), (0,)), ((), ())),
                           preferred_element_type=jnp.float32)


def _tc_mlp_body(cc_m, dd_m,
                 ec_w1, ec_b1, ec_w2, ec_b2, ec_w3, ec_b3,
                 dc_w1, dc_b1, dc_w2, dc_b2, dc_w3, dc_b3,
                 ed_w1, ed_b1, ed_w2, ed_b2, ed_w3, ed_b3,
                 sd_w1, sd_b1, sd_w2, sd_b2, sd_w3, sd_b3,
                 xc_ref, xd_ref):
    relu = lambda x: jnp.maximum(x, 0.0)
    sig = lambda x: 1.0 / (1.0 + jnp.exp(-x))

    x_cir = relu(_mm(cc_m[...], ec_w1[...]) + ec_b1[...])
    x_cir = relu(_mm(x_cir, ec_w2[...]) + ec_b2[...])
    x_cir = relu(_mm(x_cir, ec_w3[...]) + ec_b3[...])
    x_cir = relu(_mm(x_cir, dc_w1[...]) + dc_b1[...])
    x_cir = relu(_mm(x_cir, dc_w2[...]) + dc_b2[...])
    xc_ref[...] = sig(_mm(x_cir, dc_w3[...]) + dc_b3[...])

    x_dis = relu(_mm(dd_m[...], ed_w1[...]) + ed_b1[...])
    x_dis = relu(_mm(x_dis, ed_w2[...]) + ed_b2[...])
    x_dis = relu(_mm(x_dis, ed_w3[...]) + ed_b3[...])
    x_dis = relu(_mm(x_dis, sd_w1[...]) + sd_b1[...])
    x_dis = relu(_mm(x_dis, sd_w2[...]) + sd_b2[...])
    xd_ref[...] = relu(_mm(x_dis, sd_w3[...]) + sd_b3[...])


def _tc_gcn_body(x_cir, x_dis, adj_cc, adj_dd,
                 gc1_w, gc1_b, gc2_w, gc2_b,
                 gd1_w, gd1_b, gd2_w, gd2_b,
                 wc, bc, wd, bd,
                 out_ref, cir_ref, dis_ref):
    relu = lambda x: jnp.maximum(x, 0.0)

    def norm_adj(adj, nn):
        rows = lax.broadcasted_iota(jnp.int32, (nn, nn), 0)
        cols = lax.broadcasted_iota(jnp.int32, (nn, nn), 1)
        a = adj[...] + jnp.where(rows == cols, 1.0, 0.0)
        deg = jnp.sum(a, axis=1, keepdims=True)
        dinv = jnp.where(deg > 0, lax.rsqrt(jnp.where(deg > 0, deg, 1.0)), 0.0)
        return a, dinv

    a_cc, dinv_cc = norm_adj(adj_cc, N_CIR)
    a_dd, dinv_dd = norm_adj(adj_dd, N_DIS)

    def gcn(a, dinv, x, w, b):
        h = _mm(x, w[...]) * dinv
        return relu(_mm(a, h) * dinv + b[...])

    f1c = gcn(a_cc, dinv_cc, x_cir[...], gc1_w, gc1_b)
    f2c = gcn(a_cc, dinv_cc, f1c, gc2_w, gc2_b)
    f1d = gcn(a_dd, dinv_dd, x_dis[...], gd1_w, gd1_b)
    f2d = gcn(a_dd, dinv_dd, f1d, gd2_w, gd2_b)

    def _mmT(a, b):
        return lax.dot_general(a, b, (((1,), (1,)), ((), ())),
                               preferred_element_type=jnp.float32)

    cir = _mmT(f1c, wc[:, 0:128]) + _mmT(f2c, wc[:, 128:256]) + bc[...]
    dis = _mmT(f1d, wd[:, 0:128]) + _mmT(f2d, wd[:, 128:256]) + bd[...]

    cir_ref[...] = cir
    dis_ref[...] = dis
    out_ref[...] = lax.dot_general(cir, dis, (((1,), (1,)), ((), ())),
                                   preferred_element_type=jnp.float32)


def kernel(cc_data_matrix, dd_data_matrix, cc_edges, dd_edges,
           ec_w1, ec_b1, ec_w2, ec_b2, ec_w3, ec_b3,
           dc_w1, dc_b1, dc_w2, dc_b2, dc_w3, dc_b3,
           ed_w1, ed_b1, ed_w2, ed_b2, ed_w3, ed_b3,
           sd_w1, sd_b1, sd_w2, sd_b2, sd_w3, sd_b3,
           gc1_w, gc1_b, gc2_w, gc2_b,
           gd1_w, gd1_b, gd2_w, gd2_b,
           cnnc_w, cnnc_b, cnnd_w, cnnd_b):
    i32 = jnp.int32

    i = jnp.arange(NS * QE - E_CC, dtype=i32)
    j = jnp.arange(NS * QE_D - E_DD, dtype=i32)
    e_all = jnp.concatenate([
        cc_edges[0].astype(i32), i % 111, dd_edges[0].astype(i32), j % 101,
        cc_edges[1].astype(i32), jnp.full(i.shape, N_CIR, i32),
        dd_edges[1].astype(i32), jnp.full(j.shape, N_DIS, i32)])
    m_cc = cc_data_matrix.reshape(-1)
    aux = jnp.concatenate([dd_data_matrix.reshape(-1),
                           jnp.zeros((112,), jnp.float32)])
    m_tail = jnp.concatenate([m_cc[MT_OFF:], jnp.zeros((7,), jnp.float32)])

    occ, odd = _sc_build_adj()(m_cc, m_tail, aux, e_all)
    adj_cc = occ[:CC_SZ].reshape(N_CIR, N_CIR)
    adj_dd = odd[:DD_SZ].reshape(N_DIS, N_DIS)

    biases = [b.reshape(1, -1) for b in
              (ec_b1, ec_b2, ec_b3, dc_b1, dc_b2, dc_b3,
               ed_b1, ed_b2, ed_b3, sd_b1, sd_b2, sd_b3,
               gc1_b, gc2_b, gd1_b, gd2_b)]
    (ec_b1, ec_b2, ec_b3, dc_b1, dc_b2, dc_b3,
     ed_b1, ed_b2, ed_b3, sd_b1, sd_b2, sd_b3,
     gc1_b, gc2_b, gd1_b, gd2_b) = biases
    wc = cnnc_w.reshape(256, 256)
    wd = cnnd_w.reshape(256, 256)
    bc = cnnc_b.reshape(1, -1)
    bd = cnnd_b.reshape(1, -1)

    x_cir, x_dis = pl.pallas_call(
        _tc_mlp_body,
        out_shape=[
            jax.ShapeDtypeStruct((N_CIR, 64), jnp.float32),
            jax.ShapeDtypeStruct((N_DIS, 64), jnp.float32),
        ],
    )(cc_data_matrix, dd_data_matrix,
      ec_w1, ec_b1, ec_w2, ec_b2, ec_w3, ec_b3,
      dc_w1, dc_b1, dc_w2, dc_b2, dc_w3, dc_b3,
      ed_w1, ed_b1, ed_w2, ed_b2, ed_w3, ed_b3,
      sd_w1, sd_b1, sd_w2, sd_b2, sd_w3, sd_b3)

    out, cir_fea, dis_fea = pl.pallas_call(
        _tc_gcn_body,
        out_shape=[
            jax.ShapeDtypeStruct((N_CIR, N_DIS), jnp.float32),
            jax.ShapeDtypeStruct((N_CIR, 256), jnp.float32),
            jax.ShapeDtypeStruct((N_DIS, 256), jnp.float32),
        ],
    )(x_cir, x_dis, adj_cc, adj_dd,
      gc1_w, gc1_b, gc2_w, gc2_b,
      gd1_w, gd1_b, gd2_w, gd2_b,
      wc, bc, wd, bd)
    return out, cir_fea, dis_fea
